# Initial kernel scaffold; baseline (speedup 1.0000x reference)
import functools
import jax
import jax.numpy as jnp
from jax import lax
from jax.experimental import pallas as pl
from jax.experimental.pallas import tpu as pltpu
from jax.experimental.pallas import tpu_sc as plsc

N = 10000
E = 320000
F = 128
NPAD = 10240
NC = 2
NS = 16
EPT = E // (NC * NS)
C = 80
NCH = EPT // C
SLICE = NPAD // NS
RB = 1024
NEG = 0.2

f32 = jnp.float32

_mesh = plsc.VectorSubcoreMesh(core_axis_name="c", subcore_axis_name="s")


def _leaky(x):
    return jnp.where(x >= 0, x, NEG * x)


def _tc_prologue(feat_pad, W, Wattn):
    def body(f_ref, w_ref, wa_ref, h_ref, s_ref):
        x = f_ref[...]
        h = lax.dot_general(x, w_ref[...], (((1,), (1,)), ((), ())),
                            preferred_element_type=f32)
        h_ref[...] = h
        s_ref[...] = jnp.dot(h, wa_ref[...], preferred_element_type=f32)

    return pl.pallas_call(
        body,
        grid=(NPAD // RB,),
        in_specs=[
            pl.BlockSpec((RB, F), lambda i: (i, 0)),
            pl.BlockSpec((F, F), lambda i: (0, 0)),
            pl.BlockSpec((F, F), lambda i: (0, 0)),
        ],
        out_specs=[
            pl.BlockSpec((RB, F), lambda i: (i, 0)),
            pl.BlockSpec((RB, F), lambda i: (i, 0)),
        ],
        out_shape=[
            jax.ShapeDtypeStruct((NPAD, F), f32),
            jax.ShapeDtypeStruct((NPAD, F), f32),
        ],
    )(feat_pad, W, Wattn)


def _sc_edge_pass(el, er, src, dst, zvec):
    @functools.partial(
        pl.kernel,
        mesh=_mesh,
        out_type=[
            jax.ShapeDtypeStruct((E,), f32),
            jax.ShapeDtypeStruct((NC, NPAD), f32),
            jax.ShapeDtypeStruct((NC, NPAD), f32),
            jax.ShapeDtypeStruct((NC, NPAD), f32),
            jax.ShapeDtypeStruct((NC, NPAD), f32),
        ],
        scratch_types=[
            pltpu.VMEM((NPAD,), f32),
            pltpu.VMEM((NPAD,), f32),
            pltpu.VMEM((C,), jnp.int32),
            pltpu.VMEM((C,), jnp.int32),
            pltpu.VMEM((C,), f32),
            pltpu.VMEM((C,), f32),
            pltpu.VMEM_SHARED((NPAD,), f32),
            pltpu.VMEM_SHARED((NPAD,), f32),
            pltpu.VMEM_SHARED((NPAD,), f32),
            pltpu.VMEM_SHARED((NPAD,), f32),
        ],
    )
    def k(el_hbm, er_hbm, src_hbm, dst_hbm, z_hbm,
          ex_hbm, oss_hbm, osd_hbm, ods_hbm, odd_hbm,
          el_t, er_t, src_v, dst_v, ex_v, ones_v,
          acc_ss, acc_sd, acc_ds, acc_dd):
        cid = lax.axis_index("c")
        sid = lax.axis_index("s")
        base = (cid * NS + sid) * EPT
        r0 = sid * SLICE

        pltpu.sync_copy(el_hbm, el_t)
        pltpu.sync_copy(er_hbm, er_t)
        for acc in (acc_ss, acc_sd, acc_ds, acc_dd):
            pltpu.sync_copy(z_hbm.at[pl.ds(r0, SLICE)],
                            acc.at[pl.ds(r0, SLICE)])
        for kk in range(C // 16):
            ones_v[pl.ds(16 * kk, 16)] = jnp.ones((16,), f32)
        plsc.subcore_barrier()

        @pl.loop(0, NCH)
        def _(i):
            eb = base + i * C
            pltpu.sync_copy(src_hbm.at[pl.ds(eb, C)], src_v)
            pltpu.sync_copy(dst_hbm.at[pl.ds(eb, C)], dst_v)
            for kk in range(C // 16):
                sl = pl.ds(16 * kk, 16)
                s16 = src_v[sl]
                d16 = dst_v[sl]
                x = plsc.load_gather(el_t, [s16]) + plsc.load_gather(er_t, [d16])
                ex_v[sl] = jnp.exp(_leaky(x))
            pltpu.sync_copy(ex_v, ex_hbm.at[pl.ds(eb, C)])
            pltpu.sync_copy(ex_v, acc_ss.at[src_v], add=True)
            pltpu.sync_copy(ex_v, acc_sd.at[dst_v], add=True)
            pltpu.sync_copy(ones_v, acc_ds.at[src_v], add=True)
            pltpu.sync_copy(ones_v, acc_dd.at[dst_v], add=True)

        plsc.subcore_barrier()
        for acc, out in ((acc_ss, oss_hbm), (acc_sd, osd_hbm),
                         (acc_ds, ods_hbm), (acc_dd, odd_hbm)):
            pltpu.sync_copy(acc.at[pl.ds(r0, SLICE)],
                            out.at[cid, pl.ds(r0, SLICE)])

    return k(el, er, src, dst, zvec)


def _tc_mid(pss, psd, pds, pdd):
    h = NPAD // F

    def body(ss_ref, sd_ref, ds_ref, dd_ref, oss, osd, ono, oni):
        oss[...] = ss_ref[0:h, :] + ss_ref[h:2 * h, :]
        osd[...] = sd_ref[0:h, :] + sd_ref[h:2 * h, :]
        od = ds_ref[0:h, :] + ds_ref[h:2 * h, :]
        idg = dd_ref[0:h, :] + dd_ref[h:2 * h, :]
        ono[...] = lax.rsqrt(jnp.maximum(od, 1.0))
        oni[...] = jnp.sqrt(jnp.maximum(idg, 1.0))

    spec2 = pl.BlockSpec((2 * h, F), lambda: (0, 0))
    spec1 = pl.BlockSpec((h, F), lambda: (0, 0))
    return pl.pallas_call(
        body,
        grid=(),
        in_specs=[spec2] * 4,
        out_specs=[spec1] * 4,
        out_shape=[jax.ShapeDtypeStruct((h, F), f32)] * 4,
    )(pss, psd, pds, pdd)


def _sc_coeff_gather(ex, src, dst, ssrc, sdst, no, ni):
    @functools.partial(
        pl.kernel,
        mesh=_mesh,
        out_type=[jax.ShapeDtypeStruct((E,), f32)] * 3,
        scratch_types=[
            pltpu.VMEM((NPAD,), f32),
            pltpu.VMEM((NPAD,), f32),
            pltpu.VMEM((NPAD,), f32),
            pltpu.VMEM((NPAD,), f32),
            pltpu.VMEM((C,), jnp.int32),
            pltpu.VMEM((C,), jnp.int32),
            pltpu.VMEM((C,), f32),
            pltpu.VMEM((C,), f32),
            pltpu.VMEM((C,), f32),
            pltpu.VMEM((C,), f32),
        ],
    )
    def k(ex_hbm, src_hbm, dst_hbm, ss_hbm, sd_hbm, no_hbm, ni_hbm,
          p_hbm, q_hbm, r_hbm,
          ss_t, sd_t, no_t, ni_t, src_v, dst_v, ex_v, p_v, q_v, r_v):
        cid = lax.axis_index("c")
        sid = lax.axis_index("s")
        base = (cid * NS + sid) * EPT

        pltpu.sync_copy(ss_hbm, ss_t)
        pltpu.sync_copy(sd_hbm, sd_t)
        pltpu.sync_copy(no_hbm, no_t)
        pltpu.sync_copy(ni_hbm, ni_t)

        @pl.loop(0, NCH)
        def _(i):
            eb = base + i * C
            pltpu.sync_copy(src_hbm.at[pl.ds(eb, C)], src_v)
            pltpu.sync_copy(dst_hbm.at[pl.ds(eb, C)], dst_v)
            pltpu.sync_copy(ex_hbm.at[pl.ds(eb, C)], ex_v)
            for kk in range(C // 16):
                sl = pl.ds(16 * kk, 16)
                s16 = src_v[sl]
                d16 = dst_v[sl]
                ex16 = ex_v[sl]
                gs = plsc.load_gather(ss_t, [s16])
                gd = plsc.load_gather(sd_t, [d16])
                p_v[sl] = jnp.maximum(ex16 / jnp.maximum(gd, 1e-20), 1e-10)
                q_v[sl] = jnp.maximum(ex16 / jnp.maximum(gs, 1e-20), 1e-10)
                r_v[sl] = plsc.load_gather(no_t, [s16]) * plsc.load_gather(ni_t, [d16])
            pltpu.sync_copy(p_v, p_hbm.at[pl.ds(eb, C)])
            pltpu.sync_copy(q_v, q_hbm.at[pl.ds(eb, C)])
            pltpu.sync_copy(r_v, r_hbm.at[pl.ds(eb, C)])

    return k(ex, src, dst, ssrc, sdst, no, ni)


def _tc_coeff(p, q, r, sig):
    def body(p_ref, q_ref, r_ref, s_ref, o_ref):
        sg = jax.nn.sigmoid(s_ref[0, 0])
        o_ref[...] = jnp.exp(sg * jnp.log(p_ref[...]) +
                             (1.0 - sg) * jnp.log(q_ref[...])) * r_ref[...]

    rows = E // F
    spec = pl.BlockSpec((rows, F), lambda: (0, 0))
    return pl.pallas_call(
        body,
        grid=(),
        in_specs=[spec, spec, spec, pl.BlockSpec((1, 1), lambda: (0, 0))],
        out_specs=spec,
        out_shape=jax.ShapeDtypeStruct((rows, F), f32),
    )(p, q, r, sig)


def _sc_hop(x, src, dst, ahat, zmat):
    @functools.partial(
        pl.kernel,
        mesh=_mesh,
        out_type=jax.ShapeDtypeStruct((NC, NPAD, F), f32),
        scratch_types=[
            pltpu.VMEM((C,), jnp.int32),
            pltpu.VMEM((C,), jnp.int32),
            pltpu.VMEM((C,), f32),
            pltpu.VMEM((C, F), f32),
            pltpu.VMEM_SHARED((NPAD, F), f32),
            pltpu.SemaphoreType.DMA,
        ],
    )
    def k(x_hbm, src_hbm, dst_hbm, a_hbm, z_hbm, y_hbm,
          src_v, dst_v, a_v, rows_v, acc, sem):
        cid = lax.axis_index("c")
        sid = lax.axis_index("s")
        base = (cid * NS + sid) * EPT
        r0 = sid * SLICE

        pltpu.sync_copy(z_hbm.at[pl.ds(r0, SLICE)], acc.at[pl.ds(r0, SLICE)])
        plsc.subcore_barrier()

        @pl.loop(0, NCH)
        def _(i):
            eb = base + i * C
            pltpu.sync_copy(src_hbm.at[pl.ds(eb, C)], src_v)
            pltpu.sync_copy(dst_hbm.at[pl.ds(eb, C)], dst_v)
            pltpu.sync_copy(a_hbm.at[pl.ds(eb, C)], a_v)
            pltpu.async_copy(x_hbm.at[src_v], rows_v, sem).wait()

            @pl.loop(0, C)
            def _(j):
                av = a_v[j]
                for cc in range(F // 16):
                    sl = pl.ds(cc * 16, 16)
                    rows_v[j, sl] = rows_v[j, sl] * av

            pltpu.sync_copy(rows_v, acc.at[dst_v], add=True)

        plsc.subcore_barrier()
        pltpu.sync_copy(acc.at[pl.ds(r0, SLICE)],
                        y_hbm.at[cid, pl.ds(r0, SLICE)])

    return k(x, src, dst, ahat, zmat)


def _tc_add(y0, y1):
    def body(a_ref, b_ref, o_ref):
        o_ref[...] = a_ref[...] + b_ref[...]

    spec = pl.BlockSpec((RB, F), lambda i: (i, 0))
    return pl.pallas_call(
        body,
        grid=(NPAD // RB,),
        in_specs=[spec, spec],
        out_specs=spec,
        out_shape=jax.ShapeDtypeStruct((NPAD, F), f32),
    )(y0, y1)


def _tc_epilogue(h, y1, y2, y3, hl, hr):
    def body(h_ref, y1_ref, y2_ref, y3_ref, hl_ref, hr_ref, o_ref):
        hb = h_ref[...]
        b1 = y1_ref[...]
        b2 = y2_ref[...]
        b3 = y3_ref[...]
        hlb = hl_ref[...]
        hrb = hr_ref[...]
        al = jnp.sum(hb * hlb, axis=-1, keepdims=True)
        w0 = _leaky(al + jnp.sum(hb * hrb, axis=-1, keepdims=True))
        w1 = _leaky(al + jnp.sum(b1 * hrb, axis=-1, keepdims=True))
        w2 = _leaky(al + jnp.sum(b2 * hrb, axis=-1, keepdims=True))
        w3 = _leaky(al + jnp.sum(b3 * hrb, axis=-1, keepdims=True))
        m = jnp.maximum(jnp.maximum(w0, w1), jnp.maximum(w2, w3))
        e0 = jnp.exp(w0 - m)
        e1 = jnp.exp(w1 - m)
        e2 = jnp.exp(w2 - m)
        e3 = jnp.exp(w3 - m)
        s = e0 + e1 + e2 + e3
        o_ref[...] = (hb * e0 + b1 * e1 + b2 * e2 + b3 * e3) / s

    spec = pl.BlockSpec((RB, F), lambda i: (i, 0))
    vspec = pl.BlockSpec((1, F), lambda i: (0, 0))
    return pl.pallas_call(
        body,
        grid=(NPAD // RB,),
        in_specs=[spec, spec, spec, spec, vspec, vspec],
        out_specs=spec,
        out_shape=jax.ShapeDtypeStruct((NPAD, F), f32),
    )(h, y1, y2, y3, hl, hr)


def kernel(feat, edge_index, W, attn_l, attn_r, hop_attn_l, hop_attn_r, sigma):
    src = edge_index[0]
    dst = edge_index[1]
    feat_pad = jnp.pad(feat, ((0, NPAD - N), (0, 0)))

    alv = attn_l.reshape(F)
    arv = attn_r.reshape(F)
    wattn = jnp.tile(jnp.stack([alv, arv], axis=1), (1, F // 2))

    h, scal = _tc_prologue(feat_pad, W, wattn)
    el = scal[:, 0]
    er = scal[:, 1]

    zvec = jnp.zeros((NPAD,), f32)
    zmat = jnp.zeros((NPAD, F), f32)

    ex, pss, psd, pds, pdd = _sc_edge_pass(el, er, src, dst, zvec)

    rs = (NC * (NPAD // F), F)
    ssrc, sdst, no, ni = _tc_mid(pss.reshape(rs), psd.reshape(rs),
                                 pds.reshape(rs), pdd.reshape(rs))

    p, q, r = _sc_coeff_gather(ex, src, dst,
                               ssrc.reshape(NPAD), sdst.reshape(NPAD),
                               no.reshape(NPAD), ni.reshape(NPAD))

    er_ = E // F
    ahat = _tc_coeff(p.reshape(er_, F), q.reshape(er_, F),
                     r.reshape(er_, F), sigma.reshape(1, 1)).reshape(E)

    x = h
    ys = []
    for _ in range(3):
        yp = _sc_hop(x, src, dst, ahat, zmat)
        x = _tc_add(yp[0], yp[1])
        ys.append(x)

    rst = _tc_epilogue(h, ys[0], ys[1], ys[2],
                       hop_attn_l.reshape(1, F), hop_attn_r.reshape(1, F))
    return rst[:N].reshape(N, 1, F)


# speedup vs baseline: 11.1257x; 11.1257x over previous
import functools
import jax
import jax.numpy as jnp
from jax import lax
from jax.experimental import pallas as pl
from jax.experimental.pallas import tpu as pltpu
from jax.experimental.pallas import tpu_sc as plsc

N = 10000
E = 320000
F = 128
NPAD = 10240
NC = 2
NS = 16
EPT = E // (NC * NS)
C = 80
NCH = EPT // C
SLICE = NPAD // NS
RB = 1024
NEG = 0.2

f32 = jnp.float32

@functools.cache
def _sc_mesh():
    return plsc.VectorSubcoreMesh(core_axis_name="c", subcore_axis_name="s")


@functools.cache
def _sc_params():
    import dataclasses
    cp = pltpu.CompilerParams()
    if "needs_layout_passes" in pltpu.CompilerParams.__dataclass_fields__:
        cp = dataclasses.replace(cp, needs_layout_passes=False)
    return cp


def _leaky(x):
    return jnp.where(x >= 0, x, NEG * x)


def _tc_prologue(feat_pad, W, Wattn):
    def body(f_ref, w_ref, wa_ref, h_ref, s_ref):
        x = f_ref[...]
        h = lax.dot_general(x, w_ref[...], (((1,), (1,)), ((), ())),
                            preferred_element_type=f32)
        h_ref[...] = h
        s_ref[...] = jnp.dot(h, wa_ref[...], preferred_element_type=f32)

    return pl.pallas_call(
        body,
        grid=(NPAD // RB,),
        in_specs=[
            pl.BlockSpec((RB, F), lambda i: (i, 0)),
            pl.BlockSpec((F, F), lambda i: (0, 0)),
            pl.BlockSpec((F, F), lambda i: (0, 0)),
        ],
        out_specs=[
            pl.BlockSpec((RB, F), lambda i: (i, 0)),
            pl.BlockSpec((RB, F), lambda i: (i, 0)),
        ],
        out_shape=[
            jax.ShapeDtypeStruct((NPAD, F), f32),
            jax.ShapeDtypeStruct((NPAD, F), f32),
        ],
    )(feat_pad, W, Wattn)


def _sc_edge_pass(el, er, src, dst, zvec):
    @functools.partial(
        pl.kernel,
        mesh=_sc_mesh(),
        compiler_params=_sc_params(),
        out_type=[
            jax.ShapeDtypeStruct((E,), f32),
            jax.ShapeDtypeStruct((NC, NPAD), f32),
            jax.ShapeDtypeStruct((NC, NPAD), f32),
            jax.ShapeDtypeStruct((NC, NPAD), f32),
            jax.ShapeDtypeStruct((NC, NPAD), f32),
        ],
        scratch_types=[
            pltpu.VMEM((NPAD,), f32),
            pltpu.VMEM((NPAD,), f32),
            pltpu.VMEM((C,), jnp.int32),
            pltpu.VMEM((C,), jnp.int32),
            pltpu.VMEM((C,), f32),
            pltpu.VMEM((C,), f32),
            pltpu.VMEM_SHARED((NPAD,), f32),
            pltpu.VMEM_SHARED((NPAD,), f32),
            pltpu.VMEM_SHARED((NPAD,), f32),
            pltpu.VMEM_SHARED((NPAD,), f32),
        ],
    )
    def k(el_hbm, er_hbm, src_hbm, dst_hbm, z_hbm,
          ex_hbm, oss_hbm, osd_hbm, ods_hbm, odd_hbm,
          el_t, er_t, src_v, dst_v, ex_v, ones_v,
          acc_ss, acc_sd, acc_ds, acc_dd):
        cid = lax.axis_index("c")
        sid = lax.axis_index("s")
        base = (cid * NS + sid) * EPT
        r0 = sid * SLICE

        pltpu.sync_copy(el_hbm, el_t)
        pltpu.sync_copy(er_hbm, er_t)
        for acc in (acc_ss, acc_sd, acc_ds, acc_dd):
            pltpu.sync_copy(z_hbm.at[pl.ds(r0, SLICE)],
                            acc.at[pl.ds(r0, SLICE)])
        for kk in range(C // 16):
            ones_v[pl.ds(16 * kk, 16)] = jnp.ones((16,), f32)
        plsc.subcore_barrier()

        @pl.loop(0, NCH)
        def _(i):
            eb = base + i * C
            pltpu.sync_copy(src_hbm.at[pl.ds(eb, C)], src_v)
            pltpu.sync_copy(dst_hbm.at[pl.ds(eb, C)], dst_v)
            for kk in range(C // 16):
                sl = pl.ds(16 * kk, 16)
                s16 = src_v[sl]
                d16 = dst_v[sl]
                x = plsc.load_gather(el_t, [s16]) + plsc.load_gather(er_t, [d16])
                ex_v[sl] = jnp.exp(_leaky(x))
            pltpu.sync_copy(ex_v, ex_hbm.at[pl.ds(eb, C)])
            pltpu.sync_copy(ex_v, acc_ss.at[src_v], add=True)
            pltpu.sync_copy(ex_v, acc_sd.at[dst_v], add=True)
            pltpu.sync_copy(ones_v, acc_ds.at[src_v], add=True)
            pltpu.sync_copy(ones_v, acc_dd.at[dst_v], add=True)

        plsc.subcore_barrier()
        for acc, out in ((acc_ss, oss_hbm), (acc_sd, osd_hbm),
                         (acc_ds, ods_hbm), (acc_dd, odd_hbm)):
            pltpu.sync_copy(acc.at[pl.ds(r0, SLICE)],
                            out.at[cid, pl.ds(r0, SLICE)])

    return k(el, er, src, dst, zvec)


def _tc_mid(pss, psd, pds, pdd):
    h = NPAD // F

    def body(ss_ref, sd_ref, ds_ref, dd_ref, oss, osd, ono, oni):
        oss[...] = ss_ref[0:h, :] + ss_ref[h:2 * h, :]
        osd[...] = sd_ref[0:h, :] + sd_ref[h:2 * h, :]
        od = ds_ref[0:h, :] + ds_ref[h:2 * h, :]
        idg = dd_ref[0:h, :] + dd_ref[h:2 * h, :]
        ono[...] = lax.rsqrt(jnp.maximum(od, 1.0))
        oni[...] = jnp.sqrt(jnp.maximum(idg, 1.0))

    spec2 = pl.BlockSpec((2 * h, F), lambda: (0, 0))
    spec1 = pl.BlockSpec((h, F), lambda: (0, 0))
    return pl.pallas_call(
        body,
        grid=(),
        in_specs=[spec2] * 4,
        out_specs=[spec1] * 4,
        out_shape=[jax.ShapeDtypeStruct((h, F), f32)] * 4,
    )(pss, psd, pds, pdd)


def _sc_coeff_gather(ex, src, dst, ssrc, sdst, no, ni):
    @functools.partial(
        pl.kernel,
        mesh=_sc_mesh(),
        compiler_params=_sc_params(),
        out_type=[jax.ShapeDtypeStruct((E,), f32)] * 3,
        scratch_types=[
            pltpu.VMEM((NPAD,), f32),
            pltpu.VMEM((NPAD,), f32),
            pltpu.VMEM((NPAD,), f32),
            pltpu.VMEM((NPAD,), f32),
            pltpu.VMEM((C,), jnp.int32),
            pltpu.VMEM((C,), jnp.int32),
            pltpu.VMEM((C,), f32),
            pltpu.VMEM((C,), f32),
            pltpu.VMEM((C,), f32),
            pltpu.VMEM((C,), f32),
        ],
    )
    def k(ex_hbm, src_hbm, dst_hbm, ss_hbm, sd_hbm, no_hbm, ni_hbm,
          p_hbm, q_hbm, r_hbm,
          ss_t, sd_t, no_t, ni_t, src_v, dst_v, ex_v, p_v, q_v, r_v):
        cid = lax.axis_index("c")
        sid = lax.axis_index("s")
        base = (cid * NS + sid) * EPT

        pltpu.sync_copy(ss_hbm, ss_t)
        pltpu.sync_copy(sd_hbm, sd_t)
        pltpu.sync_copy(no_hbm, no_t)
        pltpu.sync_copy(ni_hbm, ni_t)

        @pl.loop(0, NCH)
        def _(i):
            eb = base + i * C
            pltpu.sync_copy(src_hbm.at[pl.ds(eb, C)], src_v)
            pltpu.sync_copy(dst_hbm.at[pl.ds(eb, C)], dst_v)
            pltpu.sync_copy(ex_hbm.at[pl.ds(eb, C)], ex_v)
            for kk in range(C // 16):
                sl = pl.ds(16 * kk, 16)
                s16 = src_v[sl]
                d16 = dst_v[sl]
                ex16 = ex_v[sl]
                gs = plsc.load_gather(ss_t, [s16])
                gd = plsc.load_gather(sd_t, [d16])
                p_v[sl] = jnp.maximum(ex16 / jnp.maximum(gd, 1e-20), 1e-10)
                q_v[sl] = jnp.maximum(ex16 / jnp.maximum(gs, 1e-20), 1e-10)
                r_v[sl] = plsc.load_gather(no_t, [s16]) * plsc.load_gather(ni_t, [d16])
            pltpu.sync_copy(p_v, p_hbm.at[pl.ds(eb, C)])
            pltpu.sync_copy(q_v, q_hbm.at[pl.ds(eb, C)])
            pltpu.sync_copy(r_v, r_hbm.at[pl.ds(eb, C)])

    return k(ex, src, dst, ssrc, sdst, no, ni)


def _tc_coeff(p, q, r, sig):
    def body(p_ref, q_ref, r_ref, s_ref, o_ref):
        sg = jax.nn.sigmoid(s_ref[...])
        o_ref[...] = jnp.exp(sg * jnp.log(p_ref[...]) +
                             (1.0 - sg) * jnp.log(q_ref[...])) * r_ref[...]

    rows = E // F
    spec = pl.BlockSpec((rows, F), lambda: (0, 0))
    return pl.pallas_call(
        body,
        grid=(),
        in_specs=[spec, spec, spec, pl.BlockSpec((1, 1), lambda: (0, 0))],
        out_specs=spec,
        out_shape=jax.ShapeDtypeStruct((rows, F), f32),
    )(p, q, r, sig)


def _sc_hop(x, src, dst, ahat, zmat):
    @functools.partial(
        pl.kernel,
        mesh=_sc_mesh(),
        compiler_params=_sc_params(),
        out_type=jax.ShapeDtypeStruct((NC, NPAD, F), f32),
        scratch_types=[
            pltpu.VMEM((C,), jnp.int32),
            pltpu.VMEM((C,), jnp.int32),
            pltpu.VMEM((C,), f32),
            pltpu.VMEM((C, F), f32),
            pltpu.VMEM_SHARED((NPAD, F), f32),
            pltpu.SemaphoreType.DMA,
        ],
    )
    def k(x_hbm, src_hbm, dst_hbm, a_hbm, z_hbm, y_hbm,
          src_v, dst_v, a_v, rows_v, acc, sem):
        cid = lax.axis_index("c")
        sid = lax.axis_index("s")
        base = (cid * NS + sid) * EPT
        r0 = sid * SLICE

        pltpu.sync_copy(z_hbm.at[pl.ds(r0, SLICE)], acc.at[pl.ds(r0, SLICE)])
        plsc.subcore_barrier()

        @pl.loop(0, NCH)
        def _(i):
            eb = base + i * C
            pltpu.sync_copy(src_hbm.at[pl.ds(eb, C)], src_v)
            pltpu.sync_copy(dst_hbm.at[pl.ds(eb, C)], dst_v)
            pltpu.sync_copy(a_hbm.at[pl.ds(eb, C)], a_v)
            pltpu.async_copy(x_hbm.at[src_v], rows_v, sem).wait()

            @pl.loop(0, C // 16)
            def _(g):
                a16 = a_v[pl.ds(16 * g, 16)]
                for l in range(16):
                    av = a16[l]
                    for cc in range(F // 16):
                        sl = pl.ds(cc * 16, 16)
                        rows_v[16 * g + l, sl] = rows_v[16 * g + l, sl] * av

            pltpu.sync_copy(rows_v, acc.at[dst_v], add=True)

        plsc.subcore_barrier()
        pltpu.sync_copy(acc.at[pl.ds(r0, SLICE)],
                        y_hbm.at[cid, pl.ds(r0, SLICE)])

    return k(x, src, dst, ahat, zmat)


def _tc_add(y0, y1):
    def body(a_ref, b_ref, o_ref):
        o_ref[...] = a_ref[...] + b_ref[...]

    spec = pl.BlockSpec((RB, F), lambda i: (i, 0))
    return pl.pallas_call(
        body,
        grid=(NPAD // RB,),
        in_specs=[spec, spec],
        out_specs=spec,
        out_shape=jax.ShapeDtypeStruct((NPAD, F), f32),
    )(y0, y1)


def _tc_epilogue(h, y1, y2, y3, hl, hr):
    def body(h_ref, y1_ref, y2_ref, y3_ref, hl_ref, hr_ref, o_ref):
        hb = h_ref[...]
        b1 = y1_ref[...]
        b2 = y2_ref[...]
        b3 = y3_ref[...]
        hlb = hl_ref[...]
        hrb = hr_ref[...]
        al = jnp.sum(hb * hlb, axis=-1, keepdims=True)
        w0 = _leaky(al + jnp.sum(hb * hrb, axis=-1, keepdims=True))
        w1 = _leaky(al + jnp.sum(b1 * hrb, axis=-1, keepdims=True))
        w2 = _leaky(al + jnp.sum(b2 * hrb, axis=-1, keepdims=True))
        w3 = _leaky(al + jnp.sum(b3 * hrb, axis=-1, keepdims=True))
        m = jnp.maximum(jnp.maximum(w0, w1), jnp.maximum(w2, w3))
        e0 = jnp.exp(w0 - m)
        e1 = jnp.exp(w1 - m)
        e2 = jnp.exp(w2 - m)
        e3 = jnp.exp(w3 - m)
        s = e0 + e1 + e2 + e3
        o_ref[...] = (hb * e0 + b1 * e1 + b2 * e2 + b3 * e3) / s

    spec = pl.BlockSpec((RB, F), lambda i: (i, 0))
    vspec = pl.BlockSpec((1, F), lambda i: (0, 0))
    return pl.pallas_call(
        body,
        grid=(NPAD // RB,),
        in_specs=[spec, spec, spec, spec, vspec, vspec],
        out_specs=spec,
        out_shape=jax.ShapeDtypeStruct((NPAD, F), f32),
    )(h, y1, y2, y3, hl, hr)


def kernel(feat, edge_index, W, attn_l, attn_r, hop_attn_l, hop_attn_r, sigma):
    src = edge_index[0]
    dst = edge_index[1]
    feat_pad = jnp.pad(feat, ((0, NPAD - N), (0, 0)))

    alv = attn_l.reshape(F)
    arv = attn_r.reshape(F)
    wattn = jnp.tile(jnp.stack([alv, arv], axis=1), (1, F // 2))

    h, scal = _tc_prologue(feat_pad, W, wattn)
    el = scal[:, 0]
    er = scal[:, 1]

    zvec = jnp.zeros((NPAD,), f32)
    zmat = jnp.zeros((NPAD, F), f32)

    ex, pss, psd, pds, pdd = _sc_edge_pass(el, er, src, dst, zvec)

    rs = (NC * (NPAD // F), F)
    ssrc, sdst, no, ni = _tc_mid(pss.reshape(rs), psd.reshape(rs),
                                 pds.reshape(rs), pdd.reshape(rs))

    p, q, r = _sc_coeff_gather(ex, src, dst,
                               ssrc.reshape(NPAD), sdst.reshape(NPAD),
                               no.reshape(NPAD), ni.reshape(NPAD))

    er_ = E // F
    ahat = _tc_coeff(p.reshape(er_, F), q.reshape(er_, F),
                     r.reshape(er_, F), sigma.reshape(1, 1)).reshape(E)

    x = h
    ys = []
    for _ in range(3):
        yp = _sc_hop(x, src, dst, ahat, zmat)
        x = _tc_add(yp[0], yp[1])
        ys.append(x)

    rst = _tc_epilogue(h, ys[0], ys[1], ys[2],
                       hop_attn_l.reshape(1, F), hop_attn_r.reshape(1, F))
    return rst[:N].reshape(N, 1, F)


# speedup vs baseline: 19.3863x; 1.7425x over previous
import functools
import jax
import jax.numpy as jnp
from jax import lax
from jax.experimental import pallas as pl
from jax.experimental.pallas import tpu as pltpu
from jax.experimental.pallas import tpu_sc as plsc

N = 10000
E = 320000
F = 128
NPAD = 10240
NC = 2
NS = 16
EPT = E // (NC * NS)
C = 80
NCH = EPT // C
SLICE = NPAD // NS
RB = 1024
NEG = 0.2

f32 = jnp.float32

@functools.cache
def _sc_mesh():
    return plsc.VectorSubcoreMesh(core_axis_name="c", subcore_axis_name="s")


@functools.cache
def _sc_params():
    import dataclasses
    cp = pltpu.CompilerParams()
    if "needs_layout_passes" in pltpu.CompilerParams.__dataclass_fields__:
        cp = dataclasses.replace(cp, needs_layout_passes=False)
    return cp


def _leaky(x):
    return jnp.where(x >= 0, x, NEG * x)


def _tc_prologue(feat_pad, W, Wattn):
    def body(f_ref, w_ref, wa_ref, h_ref, s_ref):
        x = f_ref[...]
        h = lax.dot_general(x, w_ref[...], (((1,), (1,)), ((), ())),
                            preferred_element_type=f32)
        h_ref[...] = h
        s_ref[...] = jnp.dot(h, wa_ref[...], preferred_element_type=f32)

    return pl.pallas_call(
        body,
        grid=(NPAD // RB,),
        in_specs=[
            pl.BlockSpec((RB, F), lambda i: (i, 0)),
            pl.BlockSpec((F, F), lambda i: (0, 0)),
            pl.BlockSpec((F, F), lambda i: (0, 0)),
        ],
        out_specs=[
            pl.BlockSpec((RB, F), lambda i: (i, 0)),
            pl.BlockSpec((RB, F), lambda i: (i, 0)),
        ],
        out_shape=[
            jax.ShapeDtypeStruct((NPAD, F), f32),
            jax.ShapeDtypeStruct((NPAD, F), f32),
        ],
    )(feat_pad, W, Wattn)


def _sc_edge_pass(el, er, src, dst, zvec):
    @functools.partial(
        pl.kernel,
        mesh=_sc_mesh(),
        compiler_params=_sc_params(),
        out_type=[
            jax.ShapeDtypeStruct((E,), f32),
            jax.ShapeDtypeStruct((NC, NPAD), f32),
            jax.ShapeDtypeStruct((NC, NPAD), f32),
            jax.ShapeDtypeStruct((NC, NPAD), f32),
            jax.ShapeDtypeStruct((NC, NPAD), f32),
        ],
        scratch_types=[
            pltpu.VMEM((NPAD,), f32),
            pltpu.VMEM((NPAD,), f32),
            pltpu.VMEM((C,), jnp.int32),
            pltpu.VMEM((C,), jnp.int32),
            pltpu.VMEM((C,), f32),
            pltpu.VMEM((C,), f32),
            pltpu.VMEM_SHARED((NPAD,), f32),
            pltpu.VMEM_SHARED((NPAD,), f32),
            pltpu.VMEM_SHARED((NPAD,), f32),
            pltpu.VMEM_SHARED((NPAD,), f32),
        ],
    )
    def k(el_hbm, er_hbm, src_hbm, dst_hbm, z_hbm,
          ex_hbm, oss_hbm, osd_hbm, ods_hbm, odd_hbm,
          el_t, er_t, src_v, dst_v, ex_v, ones_v,
          acc_ss, acc_sd, acc_ds, acc_dd):
        cid = lax.axis_index("c")
        sid = lax.axis_index("s")
        base = (cid * NS + sid) * EPT
        r0 = sid * SLICE

        pltpu.sync_copy(el_hbm, el_t)
        pltpu.sync_copy(er_hbm, er_t)
        for acc in (acc_ss, acc_sd, acc_ds, acc_dd):
            pltpu.sync_copy(z_hbm.at[pl.ds(r0, SLICE)],
                            acc.at[pl.ds(r0, SLICE)])
        for kk in range(C // 16):
            ones_v[pl.ds(16 * kk, 16)] = jnp.ones((16,), f32)
        plsc.subcore_barrier()

        @pl.loop(0, NCH)
        def _(i):
            eb = base + i * C
            pltpu.sync_copy(src_hbm.at[pl.ds(eb, C)], src_v)
            pltpu.sync_copy(dst_hbm.at[pl.ds(eb, C)], dst_v)
            for kk in range(C // 16):
                sl = pl.ds(16 * kk, 16)
                s16 = src_v[sl]
                d16 = dst_v[sl]
                x = plsc.load_gather(el_t, [s16]) + plsc.load_gather(er_t, [d16])
                ex_v[sl] = jnp.exp(_leaky(x))
            pltpu.sync_copy(ex_v, ex_hbm.at[pl.ds(eb, C)])
            pltpu.sync_copy(ex_v, acc_ss.at[src_v], add=True)
            pltpu.sync_copy(ex_v, acc_sd.at[dst_v], add=True)
            pltpu.sync_copy(ones_v, acc_ds.at[src_v], add=True)
            pltpu.sync_copy(ones_v, acc_dd.at[dst_v], add=True)

        plsc.subcore_barrier()
        for acc, out in ((acc_ss, oss_hbm), (acc_sd, osd_hbm),
                         (acc_ds, ods_hbm), (acc_dd, odd_hbm)):
            pltpu.sync_copy(acc.at[pl.ds(r0, SLICE)],
                            out.at[cid, pl.ds(r0, SLICE)])

    return k(el, er, src, dst, zvec)


def _tc_mid(pss, psd, pds, pdd):
    h = NPAD // F

    def body(ss_ref, sd_ref, ds_ref, dd_ref, oss, osd, ono, oni):
        oss[...] = ss_ref[0:h, :] + ss_ref[h:2 * h, :]
        osd[...] = sd_ref[0:h, :] + sd_ref[h:2 * h, :]
        od = ds_ref[0:h, :] + ds_ref[h:2 * h, :]
        idg = dd_ref[0:h, :] + dd_ref[h:2 * h, :]
        ono[...] = lax.rsqrt(jnp.maximum(od, 1.0))
        oni[...] = jnp.sqrt(jnp.maximum(idg, 1.0))

    spec2 = pl.BlockSpec((2 * h, F), lambda: (0, 0))
    spec1 = pl.BlockSpec((h, F), lambda: (0, 0))
    return pl.pallas_call(
        body,
        grid=(),
        in_specs=[spec2] * 4,
        out_specs=[spec1] * 4,
        out_shape=[jax.ShapeDtypeStruct((h, F), f32)] * 4,
    )(pss, psd, pds, pdd)


def _sc_coeff_gather(ex, src, dst, ssrc, sdst, no, ni):
    @functools.partial(
        pl.kernel,
        mesh=_sc_mesh(),
        compiler_params=_sc_params(),
        out_type=[jax.ShapeDtypeStruct((E,), f32)] * 3,
        scratch_types=[
            pltpu.VMEM((NPAD,), f32),
            pltpu.VMEM((NPAD,), f32),
            pltpu.VMEM((NPAD,), f32),
            pltpu.VMEM((NPAD,), f32),
            pltpu.VMEM((C,), jnp.int32),
            pltpu.VMEM((C,), jnp.int32),
            pltpu.VMEM((C,), f32),
            pltpu.VMEM((C,), f32),
            pltpu.VMEM((C,), f32),
            pltpu.VMEM((C,), f32),
        ],
    )
    def k(ex_hbm, src_hbm, dst_hbm, ss_hbm, sd_hbm, no_hbm, ni_hbm,
          p_hbm, q_hbm, r_hbm,
          ss_t, sd_t, no_t, ni_t, src_v, dst_v, ex_v, p_v, q_v, r_v):
        cid = lax.axis_index("c")
        sid = lax.axis_index("s")
        base = (cid * NS + sid) * EPT

        pltpu.sync_copy(ss_hbm, ss_t)
        pltpu.sync_copy(sd_hbm, sd_t)
        pltpu.sync_copy(no_hbm, no_t)
        pltpu.sync_copy(ni_hbm, ni_t)

        @pl.loop(0, NCH)
        def _(i):
            eb = base + i * C
            pltpu.sync_copy(src_hbm.at[pl.ds(eb, C)], src_v)
            pltpu.sync_copy(dst_hbm.at[pl.ds(eb, C)], dst_v)
            pltpu.sync_copy(ex_hbm.at[pl.ds(eb, C)], ex_v)
            for kk in range(C // 16):
                sl = pl.ds(16 * kk, 16)
                s16 = src_v[sl]
                d16 = dst_v[sl]
                ex16 = ex_v[sl]
                gs = plsc.load_gather(ss_t, [s16])
                gd = plsc.load_gather(sd_t, [d16])
                p_v[sl] = jnp.maximum(ex16 / jnp.maximum(gd, 1e-20), 1e-10)
                q_v[sl] = jnp.maximum(ex16 / jnp.maximum(gs, 1e-20), 1e-10)
                r_v[sl] = plsc.load_gather(no_t, [s16]) * plsc.load_gather(ni_t, [d16])
            pltpu.sync_copy(p_v, p_hbm.at[pl.ds(eb, C)])
            pltpu.sync_copy(q_v, q_hbm.at[pl.ds(eb, C)])
            pltpu.sync_copy(r_v, r_hbm.at[pl.ds(eb, C)])

    return k(ex, src, dst, ssrc, sdst, no, ni)


def _tc_coeff(p, q, r, sig):
    def body(p_ref, q_ref, r_ref, s_ref, o_ref):
        sg = jax.nn.sigmoid(s_ref[...])
        o_ref[...] = jnp.exp(sg * jnp.log(p_ref[...]) +
                             (1.0 - sg) * jnp.log(q_ref[...])) * r_ref[...]

    rows = E // F
    spec = pl.BlockSpec((rows, F), lambda: (0, 0))
    return pl.pallas_call(
        body,
        grid=(),
        in_specs=[spec, spec, spec, pl.BlockSpec((1, 1), lambda: (0, 0))],
        out_specs=spec,
        out_shape=jax.ShapeDtypeStruct((rows, F), f32),
    )(p, q, r, sig)


def _sc_hop(x, src2, dst3, a2, zmat):
    @functools.partial(
        pl.kernel,
        mesh=_sc_mesh(),
        compiler_params=_sc_params(),
        out_type=jax.ShapeDtypeStruct((NC, NPAD, F), f32),
        scratch_types=[
            pltpu.VMEM((EPT,), jnp.int32),
            pltpu.VMEM((C,), jnp.int32),
            pltpu.VMEM((C,), jnp.int32),
            pltpu.VMEM((EPT,), f32),
            pltpu.VMEM((C, F), f32),
            pltpu.VMEM((C, F), f32),
            pltpu.VMEM_SHARED((NPAD, F), f32),
            pltpu.SemaphoreType.DMA,
            pltpu.SemaphoreType.DMA,
            pltpu.SemaphoreType.DMA,
            pltpu.SemaphoreType.DMA,
            pltpu.SemaphoreType.DMA,
            pltpu.SemaphoreType.DMA,
        ],
    )
    def k(x_hbm, src_hbm, dst_hbm, a_hbm, z_hbm, y_hbm,
          src_t, dst0, dst1, a_t, rows0, rows1, acc,
          g0, g1, s0, s1, d0, d1):
        cid = lax.axis_index("c")
        sid = lax.axis_index("s")
        wid = cid * NS + sid
        r0 = sid * SLICE

        pltpu.sync_copy(src_hbm.at[wid], src_t)
        pltpu.sync_copy(a_hbm.at[wid], a_t)
        pltpu.sync_copy(z_hbm.at[pl.ds(r0, SLICE)], acc.at[pl.ds(r0, SLICE)])
        plsc.subcore_barrier()

        def scale(rows, ch):
            @pl.loop(0, C // 16)
            def _(g):
                a16 = a_t[pl.ds(ch * C + 16 * g, 16)]
                for l in range(16):
                    av = a16[l]
                    for cc in range(F // 16):
                        sl = pl.ds(cc * 16, 16)
                        rows[16 * g + l, sl] = rows[16 * g + l, sl] * av

        def gat(ch, rows, sem):
            return pltpu.make_async_copy(
                x_hbm.at[src_t.at[pl.ds(ch * C, C)]], rows, sem)

        def dget(ch, dbuf, sem):
            return pltpu.make_async_copy(dst_hbm.at[wid * NCH + ch], dbuf, sem)

        gat(0, rows0, g0).start()
        gat(1, rows1, g1).start()
        dget(0, dst0, d0).start()
        dget(1, dst1, d1).start()

        @pl.loop(0, (NCH - 1) // 2)
        def _(i):
            c0 = 2 * i
            gat(c0, rows0, g0).wait()
            scale(rows0, c0)
            dget(c0, dst0, d0).wait()
            pltpu.async_copy(rows0, acc.at[dst0], s0, add=True)
            gat(c0 + 1, rows1, g1).wait()
            scale(rows1, c0 + 1)
            dget(c0 + 1, dst1, d1).wait()
            pltpu.async_copy(rows1, acc.at[dst1], s1, add=True)
            pltpu.make_async_copy(rows0, acc.at[dst0], s0).wait()
            gat(c0 + 2, rows0, g0).start()
            dget(c0 + 2, dst0, d0).start()
            pltpu.make_async_copy(rows1, acc.at[dst1], s1).wait()

            @pl.when(i < (NCH - 1) // 2 - 1)
            def _():
                gat(c0 + 3, rows1, g1).start()
                dget(c0 + 3, dst1, d1).start()

        last = NCH - 1
        gat(last, rows0, g0).wait()
        scale(rows0, last)
        dget(last, dst0, d0).wait()
        pltpu.sync_copy(rows0, acc.at[dst0], add=True)

        plsc.subcore_barrier()
        pltpu.sync_copy(acc.at[pl.ds(r0, SLICE)],
                        y_hbm.at[cid, pl.ds(r0, SLICE)])

    return k(x, src2, dst3, a2, zmat)


def _tc_add(y0, y1):
    def body(a_ref, b_ref, o_ref):
        o_ref[...] = a_ref[...] + b_ref[...]

    spec = pl.BlockSpec((RB, F), lambda i: (i, 0))
    return pl.pallas_call(
        body,
        grid=(NPAD // RB,),
        in_specs=[spec, spec],
        out_specs=spec,
        out_shape=jax.ShapeDtypeStruct((NPAD, F), f32),
    )(y0, y1)


def _tc_epilogue(h, y1, y2, y3, hl, hr):
    def body(h_ref, y1_ref, y2_ref, y3_ref, hl_ref, hr_ref, o_ref):
        hb = h_ref[...]
        b1 = y1_ref[...]
        b2 = y2_ref[...]
        b3 = y3_ref[...]
        hlb = hl_ref[...]
        hrb = hr_ref[...]
        al = jnp.sum(hb * hlb, axis=-1, keepdims=True)
        w0 = _leaky(al + jnp.sum(hb * hrb, axis=-1, keepdims=True))
        w1 = _leaky(al + jnp.sum(b1 * hrb, axis=-1, keepdims=True))
        w2 = _leaky(al + jnp.sum(b2 * hrb, axis=-1, keepdims=True))
        w3 = _leaky(al + jnp.sum(b3 * hrb, axis=-1, keepdims=True))
        m = jnp.maximum(jnp.maximum(w0, w1), jnp.maximum(w2, w3))
        e0 = jnp.exp(w0 - m)
        e1 = jnp.exp(w1 - m)
        e2 = jnp.exp(w2 - m)
        e3 = jnp.exp(w3 - m)
        s = e0 + e1 + e2 + e3
        o_ref[...] = (hb * e0 + b1 * e1 + b2 * e2 + b3 * e3) / s

    spec = pl.BlockSpec((RB, F), lambda i: (i, 0))
    vspec = pl.BlockSpec((1, F), lambda i: (0, 0))
    return pl.pallas_call(
        body,
        grid=(NPAD // RB,),
        in_specs=[spec, spec, spec, spec, vspec, vspec],
        out_specs=spec,
        out_shape=jax.ShapeDtypeStruct((NPAD, F), f32),
    )(h, y1, y2, y3, hl, hr)


def kernel(feat, edge_index, W, attn_l, attn_r, hop_attn_l, hop_attn_r, sigma):
    src = edge_index[0]
    dst = edge_index[1]
    feat_pad = jnp.pad(feat, ((0, NPAD - N), (0, 0)))

    alv = attn_l.reshape(F)
    arv = attn_r.reshape(F)
    wattn = jnp.tile(jnp.stack([alv, arv], axis=1), (1, F // 2))

    h, scal = _tc_prologue(feat_pad, W, wattn)
    el = scal[:, 0]
    er = scal[:, 1]

    zvec = jnp.zeros((NPAD,), f32)
    zmat = jnp.zeros((NPAD, F), f32)

    ex, pss, psd, pds, pdd = _sc_edge_pass(el, er, src, dst, zvec)

    rs = (NC * (NPAD // F), F)
    ssrc, sdst, no, ni = _tc_mid(pss.reshape(rs), psd.reshape(rs),
                                 pds.reshape(rs), pdd.reshape(rs))

    p, q, r = _sc_coeff_gather(ex, src, dst,
                               ssrc.reshape(NPAD), sdst.reshape(NPAD),
                               no.reshape(NPAD), ni.reshape(NPAD))

    er_ = E // F
    ahat = _tc_coeff(p.reshape(er_, F), q.reshape(er_, F),
                     r.reshape(er_, F), sigma.reshape(1, 1)).reshape(E)

    src2 = src.reshape(NC * NS, EPT)
    dst3 = dst.reshape(NC * NS * NCH, C)
    a2 = ahat.reshape(NC * NS, EPT)

    x = h
    ys = []
    for _ in range(3):
        yp = _sc_hop(x, src2, dst3, a2, zmat)
        x = _tc_add(yp[0], yp[1])
        ys.append(x)

    rst = _tc_epilogue(h, ys[0], ys[1], ys[2],
                       hop_attn_l.reshape(1, F), hop_attn_r.reshape(1, F))
    return rst[:N].reshape(N, 1, F)


# speedup vs baseline: 26.8500x; 1.3850x over previous
import functools
import jax
import jax.numpy as jnp
from jax import lax
from jax.experimental import pallas as pl
from jax.experimental.pallas import tpu as pltpu
from jax.experimental.pallas import tpu_sc as plsc

N = 10000
E = 320000
F = 128
NPAD = 10240
NC = 2
NS = 16
EPT = E // (NC * NS)
C = 80
NCH = EPT // C
SLICE = NPAD // NS
RB = 1024
NEG = 0.2

f32 = jnp.float32

@functools.cache
def _sc_mesh():
    return plsc.VectorSubcoreMesh(core_axis_name="c", subcore_axis_name="s")


@functools.cache
def _sc_params():
    import dataclasses
    cp = pltpu.CompilerParams()
    if "needs_layout_passes" in pltpu.CompilerParams.__dataclass_fields__:
        cp = dataclasses.replace(cp, needs_layout_passes=False)
    return cp


def _leaky(x):
    return jnp.where(x >= 0, x, NEG * x)


def _tc_prologue(feat_pad, W, Wattn):
    def body(f_ref, w_ref, wa_ref, h_ref, s_ref):
        x = f_ref[...]
        h = lax.dot_general(x, w_ref[...], (((1,), (1,)), ((), ())),
                            preferred_element_type=f32)
        h_ref[...] = h
        s_ref[...] = jnp.dot(h, wa_ref[...], preferred_element_type=f32)

    return pl.pallas_call(
        body,
        grid=(NPAD // RB,),
        in_specs=[
            pl.BlockSpec((RB, F), lambda i: (i, 0)),
            pl.BlockSpec((F, F), lambda i: (0, 0)),
            pl.BlockSpec((F, F), lambda i: (0, 0)),
        ],
        out_specs=[
            pl.BlockSpec((RB, F), lambda i: (i, 0)),
            pl.BlockSpec((RB, F), lambda i: (i, 0)),
        ],
        out_shape=[
            jax.ShapeDtypeStruct((NPAD, F), f32),
            jax.ShapeDtypeStruct((NPAD, F), f32),
        ],
    )(feat_pad, W, Wattn)


def _sc_edge_pass(el, er, src, dst, zvec):
    @functools.partial(
        pl.kernel,
        mesh=_sc_mesh(),
        compiler_params=_sc_params(),
        out_type=[
            jax.ShapeDtypeStruct((E,), f32),
            jax.ShapeDtypeStruct((NC, NPAD), f32),
            jax.ShapeDtypeStruct((NC, NPAD), f32),
            jax.ShapeDtypeStruct((NC, NPAD), f32),
            jax.ShapeDtypeStruct((NC, NPAD), f32),
        ],
        scratch_types=[
            pltpu.VMEM((NPAD,), f32),
            pltpu.VMEM((NPAD,), f32),
            pltpu.VMEM((C,), jnp.int32),
            pltpu.VMEM((C,), jnp.int32),
            pltpu.VMEM((C,), jnp.int32),
            pltpu.VMEM((C,), jnp.int32),
            pltpu.VMEM((EPT,), f32),
            pltpu.VMEM((C,), f32),
            pltpu.VMEM_SHARED((NPAD,), f32),
            pltpu.VMEM_SHARED((NPAD,), f32),
            pltpu.VMEM_SHARED((NPAD,), f32),
            pltpu.VMEM_SHARED((NPAD,), f32),
            pltpu.SemaphoreType.DMA,
            pltpu.SemaphoreType.DMA,
            pltpu.SemaphoreType.DMA,
            pltpu.SemaphoreType.DMA,
            pltpu.SemaphoreType.DMA,
            pltpu.SemaphoreType.DMA,
        ],
    )
    def k(el_hbm, er_hbm, src_hbm, dst_hbm, z_hbm,
          ex_hbm, oss_hbm, osd_hbm, ods_hbm, odd_hbm,
          el_t, er_t, src_v, dst_v, src_w, dst_w, ex_a, ones_v,
          acc_ss, acc_sd, acc_ds, acc_dd,
          sa0, sb0, sa1, sb1, t0, t1):
        cid = lax.axis_index("c")
        sid = lax.axis_index("s")
        wid = cid * NS + sid
        base = wid * EPT
        r0 = sid * SLICE

        pltpu.sync_copy(el_hbm, el_t)
        pltpu.sync_copy(er_hbm, er_t)
        for acc in (acc_ss, acc_sd, acc_ds, acc_dd):
            pltpu.sync_copy(z_hbm.at[pl.ds(r0, SLICE)],
                            acc.at[pl.ds(r0, SLICE)])
        for kk in range(C // 16):
            ones_v[pl.ds(16 * kk, 16)] = jnp.ones((16,), f32)
        plsc.subcore_barrier()

        def iget(hbm, ch, buf, sem):
            return pltpu.make_async_copy(hbm.at[wid * NCH + ch], buf, sem)

        def compute(ch, sbuf, dbuf):
            @pl.loop(0, C // 16)
            def _(kk):
                sl = pl.ds(16 * kk, 16)
                x = (plsc.load_gather(el_t, [sbuf[sl]]) +
                     plsc.load_gather(er_t, [dbuf[sl]]))
                ex_a[pl.ds(ch * C + 16 * kk, 16)] = jnp.exp(_leaky(x))

        def streams(ch, sbuf, dbuf, sem):
            exs = ex_a.at[pl.ds(ch * C, C)]
            pltpu.async_copy(exs, acc_ss.at[sbuf], sem, add=True)
            pltpu.async_copy(exs, acc_sd.at[dbuf], sem, add=True)
            pltpu.async_copy(ones_v, acc_ds.at[sbuf], sem, add=True)
            pltpu.async_copy(ones_v, acc_dd.at[dbuf], sem, add=True)

        def drain(ch, sbuf, dbuf, sem):
            exs = ex_a.at[pl.ds(ch * C, C)]
            pltpu.make_async_copy(exs, acc_ss.at[sbuf], sem).wait()
            pltpu.make_async_copy(exs, acc_sd.at[dbuf], sem).wait()
            pltpu.make_async_copy(ones_v, acc_ds.at[sbuf], sem).wait()
            pltpu.make_async_copy(ones_v, acc_dd.at[dbuf], sem).wait()

        iget(src_hbm, 0, src_v, sa0).start()
        iget(dst_hbm, 0, dst_v, sb0).start()
        iget(src_hbm, 1, src_w, sa1).start()
        iget(dst_hbm, 1, dst_w, sb1).start()

        @pl.loop(0, (NCH - 1) // 2)
        def _(i):
            c0 = 2 * i
            iget(src_hbm, c0, src_v, sa0).wait()
            iget(dst_hbm, c0, dst_v, sb0).wait()
            compute(c0, src_v, dst_v)
            streams(c0, src_v, dst_v, t0)
            iget(src_hbm, c0 + 1, src_w, sa1).wait()
            iget(dst_hbm, c0 + 1, dst_w, sb1).wait()
            compute(c0 + 1, src_w, dst_w)
            streams(c0 + 1, src_w, dst_w, t1)
            drain(c0, src_v, dst_v, t0)
            iget(src_hbm, c0 + 2, src_v, sa0).start()
            iget(dst_hbm, c0 + 2, dst_v, sb0).start()
            drain(c0 + 1, src_w, dst_w, t1)

            @pl.when(i < (NCH - 1) // 2 - 1)
            def _():
                iget(src_hbm, c0 + 3, src_w, sa1).start()
                iget(dst_hbm, c0 + 3, dst_w, sb1).start()

        last = NCH - 1
        iget(src_hbm, last, src_v, sa0).wait()
        iget(dst_hbm, last, dst_v, sb0).wait()
        compute(last, src_v, dst_v)
        streams(last, src_v, dst_v, t0)
        drain(last, src_v, dst_v, t0)
        pltpu.sync_copy(ex_a, ex_hbm.at[pl.ds(base, EPT)])

        plsc.subcore_barrier()
        for acc, out in ((acc_ss, oss_hbm), (acc_sd, osd_hbm),
                         (acc_ds, ods_hbm), (acc_dd, odd_hbm)):
            pltpu.sync_copy(acc.at[pl.ds(r0, SLICE)],
                            out.at[cid, pl.ds(r0, SLICE)])

    return k(el, er, src, dst, zvec)


def _tc_mid(pss, psd, pds, pdd):
    h = NPAD // F

    def body(ss_ref, sd_ref, ds_ref, dd_ref, oss, osd, ono, oni):
        oss[...] = ss_ref[0:h, :] + ss_ref[h:2 * h, :]
        osd[...] = sd_ref[0:h, :] + sd_ref[h:2 * h, :]
        od = ds_ref[0:h, :] + ds_ref[h:2 * h, :]
        idg = dd_ref[0:h, :] + dd_ref[h:2 * h, :]
        ono[...] = lax.rsqrt(jnp.maximum(od, 1.0))
        oni[...] = jnp.sqrt(jnp.maximum(idg, 1.0))

    spec2 = pl.BlockSpec((2 * h, F), lambda: (0, 0))
    spec1 = pl.BlockSpec((h, F), lambda: (0, 0))
    return pl.pallas_call(
        body,
        grid=(),
        in_specs=[spec2] * 4,
        out_specs=[spec1] * 4,
        out_shape=[jax.ShapeDtypeStruct((h, F), f32)] * 4,
    )(pss, psd, pds, pdd)


def _sc_coeff_gather(ex2, src2, dst2, ssrc, sdst, no, ni):
    NW = NC * NS

    @functools.partial(
        pl.kernel,
        mesh=_sc_mesh(),
        compiler_params=_sc_params(),
        out_type=[jax.ShapeDtypeStruct((NW, EPT), f32)] * 3,
        scratch_types=[
            pltpu.VMEM((NPAD,), f32),
            pltpu.VMEM((NPAD,), f32),
            pltpu.VMEM((NPAD,), f32),
            pltpu.VMEM((NPAD,), f32),
            pltpu.VMEM((EPT,), jnp.int32),
            pltpu.VMEM((EPT,), jnp.int32),
            pltpu.VMEM((EPT,), f32),
            pltpu.VMEM((EPT,), f32),
            pltpu.VMEM((EPT,), f32),
            pltpu.VMEM((EPT,), f32),
        ],
    )
    def k(ex_hbm, src_hbm, dst_hbm, ss_hbm, sd_hbm, no_hbm, ni_hbm,
          p_hbm, q_hbm, r_hbm,
          ss_t, sd_t, no_t, ni_t, src_a, dst_a, ex_a, p_a, q_a, r_a):
        cid = lax.axis_index("c")
        sid = lax.axis_index("s")
        wid = cid * NS + sid

        pltpu.sync_copy(ss_hbm, ss_t)
        pltpu.sync_copy(sd_hbm, sd_t)
        pltpu.sync_copy(no_hbm, no_t)
        pltpu.sync_copy(ni_hbm, ni_t)
        pltpu.sync_copy(src_hbm.at[wid], src_a)
        pltpu.sync_copy(dst_hbm.at[wid], dst_a)
        pltpu.sync_copy(ex_hbm.at[wid], ex_a)

        @pl.loop(0, EPT // 16)
        def _(t):
            sl = pl.ds(16 * t, 16)
            s16 = src_a[sl]
            d16 = dst_a[sl]
            ex16 = ex_a[sl]
            gs = plsc.load_gather(ss_t, [s16])
            gd = plsc.load_gather(sd_t, [d16])
            p_a[sl] = jnp.maximum(ex16 / jnp.maximum(gd, 1e-20), 1e-10)
            q_a[sl] = jnp.maximum(ex16 / jnp.maximum(gs, 1e-20), 1e-10)
            r_a[sl] = plsc.load_gather(no_t, [s16]) * plsc.load_gather(ni_t, [d16])

        pltpu.sync_copy(p_a, p_hbm.at[wid])
        pltpu.sync_copy(q_a, q_hbm.at[wid])
        pltpu.sync_copy(r_a, r_hbm.at[wid])

    return k(ex2, src2, dst2, ssrc, sdst, no, ni)


def _tc_coeff(p, q, r, sig):
    def body(p_ref, q_ref, r_ref, s_ref, o_ref):
        sg = jax.nn.sigmoid(s_ref[...])
        o_ref[...] = jnp.exp(sg * jnp.log(p_ref[...]) +
                             (1.0 - sg) * jnp.log(q_ref[...])) * r_ref[...]

    rows = E // F
    spec = pl.BlockSpec((rows, F), lambda: (0, 0))
    return pl.pallas_call(
        body,
        grid=(),
        in_specs=[spec, spec, spec, pl.BlockSpec((1, 1), lambda: (0, 0))],
        out_specs=spec,
        out_shape=jax.ShapeDtypeStruct((rows, F), f32),
    )(p, q, r, sig)


def _sc_hop(x, src2, dst3, a2, zmat):
    @functools.partial(
        pl.kernel,
        mesh=_sc_mesh(),
        compiler_params=_sc_params(),
        out_type=jax.ShapeDtypeStruct((NC, NPAD, F), f32),
        scratch_types=[
            pltpu.VMEM((EPT,), jnp.int32),
            pltpu.VMEM((C,), jnp.int32),
            pltpu.VMEM((C,), jnp.int32),
            pltpu.VMEM((EPT,), f32),
            pltpu.VMEM((C, F), f32),
            pltpu.VMEM((C, F), f32),
            pltpu.VMEM_SHARED((NPAD, F), f32),
            pltpu.SemaphoreType.DMA,
            pltpu.SemaphoreType.DMA,
            pltpu.SemaphoreType.DMA,
            pltpu.SemaphoreType.DMA,
            pltpu.SemaphoreType.DMA,
            pltpu.SemaphoreType.DMA,
        ],
    )
    def k(x_hbm, src_hbm, dst_hbm, a_hbm, z_hbm, y_hbm,
          src_t, dst0, dst1, a_t, rows0, rows1, acc,
          g0, g1, s0, s1, d0, d1):
        cid = lax.axis_index("c")
        sid = lax.axis_index("s")
        wid = cid * NS + sid
        r0 = sid * SLICE

        pltpu.sync_copy(src_hbm.at[wid], src_t)
        pltpu.sync_copy(a_hbm.at[wid], a_t)
        pltpu.sync_copy(z_hbm.at[pl.ds(r0, SLICE)], acc.at[pl.ds(r0, SLICE)])
        plsc.subcore_barrier()

        def scale(rows, ch):
            @pl.loop(0, C // 16)
            def _(g):
                a16 = a_t[pl.ds(ch * C + 16 * g, 16)]
                for l in range(16):
                    av = a16[l]
                    for cc in range(F // 16):
                        sl = pl.ds(cc * 16, 16)
                        rows[16 * g + l, sl] = rows[16 * g + l, sl] * av

        def gat(ch, rows, sem):
            return pltpu.make_async_copy(
                x_hbm.at[src_t.at[pl.ds(ch * C, C)]], rows, sem)

        def dget(ch, dbuf, sem):
            return pltpu.make_async_copy(dst_hbm.at[wid * NCH + ch], dbuf, sem)

        gat(0, rows0, g0).start()
        gat(1, rows1, g1).start()
        dget(0, dst0, d0).start()
        dget(1, dst1, d1).start()

        @pl.loop(0, (NCH - 1) // 2)
        def _(i):
            c0 = 2 * i
            gat(c0, rows0, g0).wait()
            scale(rows0, c0)
            dget(c0, dst0, d0).wait()
            pltpu.async_copy(rows0, acc.at[dst0], s0, add=True)
            gat(c0 + 1, rows1, g1).wait()
            scale(rows1, c0 + 1)
            dget(c0 + 1, dst1, d1).wait()
            pltpu.async_copy(rows1, acc.at[dst1], s1, add=True)
            pltpu.make_async_copy(rows0, acc.at[dst0], s0).wait()
            gat(c0 + 2, rows0, g0).start()
            dget(c0 + 2, dst0, d0).start()
            pltpu.make_async_copy(rows1, acc.at[dst1], s1).wait()

            @pl.when(i < (NCH - 1) // 2 - 1)
            def _():
                gat(c0 + 3, rows1, g1).start()
                dget(c0 + 3, dst1, d1).start()

        last = NCH - 1
        gat(last, rows0, g0).wait()
        scale(rows0, last)
        dget(last, dst0, d0).wait()
        pltpu.sync_copy(rows0, acc.at[dst0], add=True)

        plsc.subcore_barrier()
        pltpu.sync_copy(acc.at[pl.ds(r0, SLICE)],
                        y_hbm.at[cid, pl.ds(r0, SLICE)])

    return k(x, src2, dst3, a2, zmat)


def _tc_add(y0, y1):
    def body(a_ref, b_ref, o_ref):
        o_ref[...] = a_ref[...] + b_ref[...]

    spec = pl.BlockSpec((RB, F), lambda i: (i, 0))
    return pl.pallas_call(
        body,
        grid=(NPAD // RB,),
        in_specs=[spec, spec],
        out_specs=spec,
        out_shape=jax.ShapeDtypeStruct((NPAD, F), f32),
    )(y0, y1)


def _tc_epilogue(h, y1, y2, y3, hl, hr):
    def body(h_ref, y1_ref, y2_ref, y3_ref, hl_ref, hr_ref, o_ref):
        hb = h_ref[...]
        b1 = y1_ref[...]
        b2 = y2_ref[...]
        b3 = y3_ref[...]
        hlb = hl_ref[...]
        hrb = hr_ref[...]
        al = jnp.sum(hb * hlb, axis=-1, keepdims=True)
        w0 = _leaky(al + jnp.sum(hb * hrb, axis=-1, keepdims=True))
        w1 = _leaky(al + jnp.sum(b1 * hrb, axis=-1, keepdims=True))
        w2 = _leaky(al + jnp.sum(b2 * hrb, axis=-1, keepdims=True))
        w3 = _leaky(al + jnp.sum(b3 * hrb, axis=-1, keepdims=True))
        m = jnp.maximum(jnp.maximum(w0, w1), jnp.maximum(w2, w3))
        e0 = jnp.exp(w0 - m)
        e1 = jnp.exp(w1 - m)
        e2 = jnp.exp(w2 - m)
        e3 = jnp.exp(w3 - m)
        s = e0 + e1 + e2 + e3
        o_ref[...] = (hb * e0 + b1 * e1 + b2 * e2 + b3 * e3) / s

    spec = pl.BlockSpec((RB, F), lambda i: (i, 0))
    vspec = pl.BlockSpec((1, F), lambda i: (0, 0))
    return pl.pallas_call(
        body,
        grid=(NPAD // RB,),
        in_specs=[spec, spec, spec, spec, vspec, vspec],
        out_specs=spec,
        out_shape=jax.ShapeDtypeStruct((NPAD, F), f32),
    )(h, y1, y2, y3, hl, hr)


def kernel(feat, edge_index, W, attn_l, attn_r, hop_attn_l, hop_attn_r, sigma):
    src = edge_index[0]
    dst = edge_index[1]
    feat_pad = jnp.pad(feat, ((0, NPAD - N), (0, 0)))

    alv = attn_l.reshape(F)
    arv = attn_r.reshape(F)
    wattn = jnp.tile(jnp.stack([alv, arv], axis=1), (1, F // 2))

    h, scal = _tc_prologue(feat_pad, W, wattn)
    el = scal[:, 0]
    er = scal[:, 1]

    zvec = jnp.zeros((NPAD,), f32)
    zmat = jnp.zeros((NPAD, F), f32)

    srcC = src.reshape(NC * NS * NCH, C)
    dstC = dst.reshape(NC * NS * NCH, C)
    src2 = src.reshape(NC * NS, EPT)
    dst2 = dst.reshape(NC * NS, EPT)

    ex, pss, psd, pds, pdd = _sc_edge_pass(el, er, srcC, dstC, zvec)

    rs = (NC * (NPAD // F), F)
    ssrc, sdst, no, ni = _tc_mid(pss.reshape(rs), psd.reshape(rs),
                                 pds.reshape(rs), pdd.reshape(rs))

    p, q, r = _sc_coeff_gather(ex.reshape(NC * NS, EPT), src2, dst2,
                               ssrc.reshape(NPAD), sdst.reshape(NPAD),
                               no.reshape(NPAD), ni.reshape(NPAD))

    er_ = E // F
    ahat = _tc_coeff(p.reshape(er_, F), q.reshape(er_, F),
                     r.reshape(er_, F), sigma.reshape(1, 1)).reshape(E)

    dst3 = dstC
    a2 = ahat.reshape(NC * NS, EPT)

    x = h
    ys = []
    for _ in range(3):
        yp = _sc_hop(x, src2, dst3, a2, zmat)
        x = _tc_add(yp[0], yp[1])
        ys.append(x)

    rst = _tc_epilogue(h, ys[0], ys[1], ys[2],
                       hop_attn_l.reshape(1, F), hop_attn_r.reshape(1, F))
    return rst[:N].reshape(N, 1, F)


# speedup vs baseline: 27.6284x; 1.0290x over previous
import functools
import jax
import jax.numpy as jnp
from jax import lax
from jax.experimental import pallas as pl
from jax.experimental.pallas import tpu as pltpu
from jax.experimental.pallas import tpu_sc as plsc

N = 10000
E = 320000
F = 128
NPAD = 10240
NC = 2
NS = 16
EPT = E // (NC * NS)
C = 80
NCH = EPT // C
SLICE = NPAD // NS
RB = 1024
NEG = 0.2

f32 = jnp.float32

@functools.cache
def _sc_mesh():
    return plsc.VectorSubcoreMesh(core_axis_name="c", subcore_axis_name="s")


@functools.cache
def _sc_params():
    import dataclasses
    cp = pltpu.CompilerParams()
    if "needs_layout_passes" in pltpu.CompilerParams.__dataclass_fields__:
        cp = dataclasses.replace(cp, needs_layout_passes=False)
    return cp


def _leaky(x):
    return jnp.where(x >= 0, x, NEG * x)


def _tc_prologue(feat_pad, W, Wattn):
    def body(f_ref, w_ref, wa_ref, h_ref, s_ref):
        x = f_ref[...]
        h = lax.dot_general(x, w_ref[...], (((1,), (1,)), ((), ())),
                            preferred_element_type=f32)
        h_ref[...] = h
        s_ref[...] = jnp.dot(h, wa_ref[...], preferred_element_type=f32)

    return pl.pallas_call(
        body,
        grid=(NPAD // RB,),
        in_specs=[
            pl.BlockSpec((RB, F), lambda i: (i, 0)),
            pl.BlockSpec((F, F), lambda i: (0, 0)),
            pl.BlockSpec((F, F), lambda i: (0, 0)),
        ],
        out_specs=[
            pl.BlockSpec((RB, F), lambda i: (i, 0)),
            pl.BlockSpec((RB, F), lambda i: (i, 0)),
        ],
        out_shape=[
            jax.ShapeDtypeStruct((NPAD, F), f32),
            jax.ShapeDtypeStruct((NPAD, F), f32),
        ],
    )(feat_pad, W, Wattn)


def _sc_edge_pass(el, er, src, dst, zvec):
    @functools.partial(
        pl.kernel,
        mesh=_sc_mesh(),
        compiler_params=_sc_params(),
        out_type=[
            jax.ShapeDtypeStruct((E,), f32),
            jax.ShapeDtypeStruct((NC, NPAD), f32),
            jax.ShapeDtypeStruct((NC, NPAD), f32),
            jax.ShapeDtypeStruct((NC, NPAD), f32),
            jax.ShapeDtypeStruct((NC, NPAD), f32),
        ],
        scratch_types=[
            pltpu.VMEM((NPAD,), f32),
            pltpu.VMEM((NPAD,), f32),
            pltpu.VMEM((C,), jnp.int32),
            pltpu.VMEM((C,), jnp.int32),
            pltpu.VMEM((C,), jnp.int32),
            pltpu.VMEM((C,), jnp.int32),
            pltpu.VMEM((EPT,), f32),
            pltpu.VMEM((C,), f32),
            pltpu.VMEM_SHARED((NPAD,), f32),
            pltpu.VMEM_SHARED((NPAD,), f32),
            pltpu.VMEM_SHARED((NPAD,), f32),
            pltpu.VMEM_SHARED((NPAD,), f32),
            pltpu.SemaphoreType.DMA,
            pltpu.SemaphoreType.DMA,
            pltpu.SemaphoreType.DMA,
            pltpu.SemaphoreType.DMA,
            pltpu.SemaphoreType.DMA,
            pltpu.SemaphoreType.DMA,
        ],
    )
    def k(el_hbm, er_hbm, src_hbm, dst_hbm, z_hbm,
          ex_hbm, oss_hbm, osd_hbm, ods_hbm, odd_hbm,
          el_t, er_t, src_v, dst_v, src_w, dst_w, ex_a, ones_v,
          acc_ss, acc_sd, acc_ds, acc_dd,
          sa0, sb0, sa1, sb1, t0, t1):
        cid = lax.axis_index("c")
        sid = lax.axis_index("s")
        wid = cid * NS + sid
        base = wid * EPT
        r0 = sid * SLICE

        pltpu.sync_copy(el_hbm, el_t)
        pltpu.sync_copy(er_hbm, er_t)
        for acc in (acc_ss, acc_sd, acc_ds, acc_dd):
            pltpu.sync_copy(z_hbm.at[pl.ds(r0, SLICE)],
                            acc.at[pl.ds(r0, SLICE)])
        for kk in range(C // 16):
            ones_v[pl.ds(16 * kk, 16)] = jnp.ones((16,), f32)
        plsc.subcore_barrier()

        def iget(hbm, ch, buf, sem):
            return pltpu.make_async_copy(hbm.at[wid * NCH + ch], buf, sem)

        def compute(ch, sbuf, dbuf):
            @pl.loop(0, C // 16)
            def _(kk):
                sl = pl.ds(16 * kk, 16)
                x = (plsc.load_gather(el_t, [sbuf[sl]]) +
                     plsc.load_gather(er_t, [dbuf[sl]]))
                ex_a[pl.ds(ch * C + 16 * kk, 16)] = jnp.exp(_leaky(x))

        def streams(ch, sbuf, dbuf, sem):
            exs = ex_a.at[pl.ds(ch * C, C)]
            pltpu.async_copy(exs, acc_ss.at[sbuf], sem, add=True)
            pltpu.async_copy(exs, acc_sd.at[dbuf], sem, add=True)
            pltpu.async_copy(ones_v, acc_ds.at[sbuf], sem, add=True)
            pltpu.async_copy(ones_v, acc_dd.at[dbuf], sem, add=True)

        def drain(ch, sbuf, dbuf, sem):
            exs = ex_a.at[pl.ds(ch * C, C)]
            pltpu.make_async_copy(exs, acc_ss.at[sbuf], sem).wait()
            pltpu.make_async_copy(exs, acc_sd.at[dbuf], sem).wait()
            pltpu.make_async_copy(ones_v, acc_ds.at[sbuf], sem).wait()
            pltpu.make_async_copy(ones_v, acc_dd.at[dbuf], sem).wait()

        iget(src_hbm, 0, src_v, sa0).start()
        iget(dst_hbm, 0, dst_v, sb0).start()
        iget(src_hbm, 1, src_w, sa1).start()
        iget(dst_hbm, 1, dst_w, sb1).start()

        @pl.loop(0, (NCH - 1) // 2)
        def _(i):
            c0 = 2 * i
            iget(src_hbm, c0, src_v, sa0).wait()
            iget(dst_hbm, c0, dst_v, sb0).wait()
            compute(c0, src_v, dst_v)
            streams(c0, src_v, dst_v, t0)
            iget(src_hbm, c0 + 1, src_w, sa1).wait()
            iget(dst_hbm, c0 + 1, dst_w, sb1).wait()
            compute(c0 + 1, src_w, dst_w)
            streams(c0 + 1, src_w, dst_w, t1)
            drain(c0, src_v, dst_v, t0)
            iget(src_hbm, c0 + 2, src_v, sa0).start()
            iget(dst_hbm, c0 + 2, dst_v, sb0).start()
            drain(c0 + 1, src_w, dst_w, t1)

            @pl.when(i < (NCH - 1) // 2 - 1)
            def _():
                iget(src_hbm, c0 + 3, src_w, sa1).start()
                iget(dst_hbm, c0 + 3, dst_w, sb1).start()

        last = NCH - 1
        iget(src_hbm, last, src_v, sa0).wait()
        iget(dst_hbm, last, dst_v, sb0).wait()
        compute(last, src_v, dst_v)
        streams(last, src_v, dst_v, t0)
        drain(last, src_v, dst_v, t0)
        pltpu.sync_copy(ex_a, ex_hbm.at[pl.ds(base, EPT)])

        plsc.subcore_barrier()
        for acc, out in ((acc_ss, oss_hbm), (acc_sd, osd_hbm),
                         (acc_ds, ods_hbm), (acc_dd, odd_hbm)):
            pltpu.sync_copy(acc.at[pl.ds(r0, SLICE)],
                            out.at[cid, pl.ds(r0, SLICE)])

    return k(el, er, src, dst, zvec)


def _tc_mid(pss, psd, pds, pdd):
    h = NPAD // F

    def body(ss_ref, sd_ref, ds_ref, dd_ref, oss, osd, ono, oni):
        oss[...] = ss_ref[0:h, :] + ss_ref[h:2 * h, :]
        osd[...] = sd_ref[0:h, :] + sd_ref[h:2 * h, :]
        od = ds_ref[0:h, :] + ds_ref[h:2 * h, :]
        idg = dd_ref[0:h, :] + dd_ref[h:2 * h, :]
        ono[...] = lax.rsqrt(jnp.maximum(od, 1.0))
        oni[...] = jnp.sqrt(jnp.maximum(idg, 1.0))

    spec2 = pl.BlockSpec((2 * h, F), lambda: (0, 0))
    spec1 = pl.BlockSpec((h, F), lambda: (0, 0))
    return pl.pallas_call(
        body,
        grid=(),
        in_specs=[spec2] * 4,
        out_specs=[spec1] * 4,
        out_shape=[jax.ShapeDtypeStruct((h, F), f32)] * 4,
    )(pss, psd, pds, pdd)


def _sc_coeff_gather(ex2, src2, dst2, ssrc, sdst, no, ni):
    NW = NC * NS

    @functools.partial(
        pl.kernel,
        mesh=_sc_mesh(),
        compiler_params=_sc_params(),
        out_type=[jax.ShapeDtypeStruct((NW, EPT), f32)] * 3,
        scratch_types=[
            pltpu.VMEM((NPAD,), f32),
            pltpu.VMEM((NPAD,), f32),
            pltpu.VMEM((NPAD,), f32),
            pltpu.VMEM((NPAD,), f32),
            pltpu.VMEM((EPT,), jnp.int32),
            pltpu.VMEM((EPT,), jnp.int32),
            pltpu.VMEM((EPT,), f32),
            pltpu.VMEM((EPT,), f32),
            pltpu.VMEM((EPT,), f32),
            pltpu.VMEM((EPT,), f32),
        ],
    )
    def k(ex_hbm, src_hbm, dst_hbm, ss_hbm, sd_hbm, no_hbm, ni_hbm,
          p_hbm, q_hbm, r_hbm,
          ss_t, sd_t, no_t, ni_t, src_a, dst_a, ex_a, p_a, q_a, r_a):
        cid = lax.axis_index("c")
        sid = lax.axis_index("s")
        wid = cid * NS + sid

        pltpu.sync_copy(ss_hbm, ss_t)
        pltpu.sync_copy(sd_hbm, sd_t)
        pltpu.sync_copy(no_hbm, no_t)
        pltpu.sync_copy(ni_hbm, ni_t)
        pltpu.sync_copy(src_hbm.at[wid], src_a)
        pltpu.sync_copy(dst_hbm.at[wid], dst_a)
        pltpu.sync_copy(ex_hbm.at[wid], ex_a)

        @pl.loop(0, EPT // 16)
        def _(t):
            sl = pl.ds(16 * t, 16)
            s16 = src_a[sl]
            d16 = dst_a[sl]
            ex16 = ex_a[sl]
            gs = plsc.load_gather(ss_t, [s16])
            gd = plsc.load_gather(sd_t, [d16])
            p_a[sl] = jnp.maximum(ex16 / jnp.maximum(gd, 1e-20), 1e-10)
            q_a[sl] = jnp.maximum(ex16 / jnp.maximum(gs, 1e-20), 1e-10)
            r_a[sl] = plsc.load_gather(no_t, [s16]) * plsc.load_gather(ni_t, [d16])

        pltpu.sync_copy(p_a, p_hbm.at[wid])
        pltpu.sync_copy(q_a, q_hbm.at[wid])
        pltpu.sync_copy(r_a, r_hbm.at[wid])

    return k(ex2, src2, dst2, ssrc, sdst, no, ni)


def _tc_coeff(p, q, r, sig):
    def body(p_ref, q_ref, r_ref, s_ref, o_ref):
        sg = jax.nn.sigmoid(s_ref[...])
        o_ref[...] = jnp.exp(sg * jnp.log(p_ref[...]) +
                             (1.0 - sg) * jnp.log(q_ref[...])) * r_ref[...]

    rows = E // F
    spec = pl.BlockSpec((rows, F), lambda: (0, 0))
    return pl.pallas_call(
        body,
        grid=(),
        in_specs=[spec, spec, spec, pl.BlockSpec((1, 1), lambda: (0, 0))],
        out_specs=spec,
        out_shape=jax.ShapeDtypeStruct((rows, F), f32),
    )(p, q, r, sig)


def _sc_hop(x, srcC, dstC, a2, zmat):
    NB = 3

    @functools.partial(
        pl.kernel,
        mesh=_sc_mesh(),
        compiler_params=_sc_params(),
        out_type=jax.ShapeDtypeStruct((NC, NPAD, F), f32),
        scratch_types=(
            [pltpu.VMEM((C,), jnp.int32)] * NB +
            [pltpu.VMEM((C,), jnp.int32)] * NB +
            [pltpu.VMEM((EPT,), f32)] +
            [pltpu.VMEM((C, F), f32)] * NB +
            [pltpu.VMEM_SHARED((NPAD, F), f32)] +
            [pltpu.SemaphoreType.DMA] * (4 * NB)
        ),
    )
    def k(x_hbm, src_hbm, dst_hbm, a_hbm, z_hbm, y_hbm, *refs):
        srcb = refs[0:NB]
        dstb = refs[NB:2 * NB]
        a_t = refs[2 * NB]
        rows = refs[2 * NB + 1:3 * NB + 1]
        acc = refs[3 * NB + 1]
        gsem = refs[3 * NB + 2:4 * NB + 2]
        ssem = refs[4 * NB + 2:5 * NB + 2]
        isem = refs[5 * NB + 2:6 * NB + 2]
        dsem = refs[6 * NB + 2:7 * NB + 2]

        cid = lax.axis_index("c")
        sid = lax.axis_index("s")
        wid = cid * NS + sid
        r0 = sid * SLICE

        pltpu.sync_copy(a_hbm.at[wid], a_t)
        pltpu.sync_copy(z_hbm.at[pl.ds(r0, SLICE)], acc.at[pl.ds(r0, SLICE)])
        plsc.subcore_barrier()

        def scale(rws, ch):
            @pl.loop(0, C // 16)
            def _(g):
                a16 = a_t[pl.ds(ch * C + 16 * g, 16)]
                for l in range(16):
                    av = a16[l]
                    for cc in range(F // 16):
                        sl = pl.ds(cc * 16, 16)
                        rws[16 * g + l, sl] = rws[16 * g + l, sl] * av

        def sget(ch, b):
            return pltpu.make_async_copy(dst_hbm.at[wid * NCH + ch],
                                         dstb[b], dsem[b])

        def iget(ch, b):
            return pltpu.make_async_copy(src_hbm.at[wid * NCH + ch],
                                         srcb[b], isem[b])

        def gat(b):
            return pltpu.make_async_copy(x_hbm.at[srcb[b]], rows[b], gsem[b])

        def sct(b):
            return pltpu.make_async_copy(rows[b], acc.at[dstb[b]], ssem[b])

        for b in range(NB):
            iget(b, b).start()
            sget(b, b).start()
        for b in range(NB):
            iget(b, b).wait()
            pltpu.async_copy(x_hbm.at[srcb[b]], rows[b], gsem[b])

        TRI = (NCH - 2) // NB

        @pl.loop(0, TRI)
        def _(i):
            c0 = NB * i
            for b in range(NB):
                gat(b).wait()
                scale(rows[b], c0 + b)
                sget(c0 + b, b).wait()
                pltpu.async_copy(rows[b], acc.at[dstb[b]], ssem[b], add=True)
            for b in range(NB):
                nc = c0 + b + NB
                sct(b).wait()

                @pl.when(nc < NCH)
                def _():
                    iget(nc, b).start()
                    sget(nc, b).start()
                    iget(nc, b).wait()
                    pltpu.async_copy(x_hbm.at[srcb[b]], rows[b], gsem[b])

        for b, ch in ((0, NCH - 2), (1, NCH - 1)):
            gat(b).wait()
            scale(rows[b], ch)
            sget(ch, b).wait()
            pltpu.sync_copy(rows[b], acc.at[dstb[b]], add=True)

        plsc.subcore_barrier()
        pltpu.sync_copy(acc.at[pl.ds(r0, SLICE)],
                        y_hbm.at[cid, pl.ds(r0, SLICE)])

    return k(x, srcC, dstC, a2, zmat)


def _tc_add(y0, y1):
    def body(a_ref, b_ref, o_ref):
        o_ref[...] = a_ref[...] + b_ref[...]

    spec = pl.BlockSpec((RB, F), lambda i: (i, 0))
    return pl.pallas_call(
        body,
        grid=(NPAD // RB,),
        in_specs=[spec, spec],
        out_specs=spec,
        out_shape=jax.ShapeDtypeStruct((NPAD, F), f32),
    )(y0, y1)


def _tc_epilogue(h, y1, y2, y3a, y3b, hl, hr):
    def body(h_ref, y1_ref, y2_ref, y3a_ref, y3b_ref, hl_ref, hr_ref, o_ref):
        hb = h_ref[...]
        b1 = y1_ref[...]
        b2 = y2_ref[...]
        b3 = y3a_ref[...] + y3b_ref[...]
        hlb = hl_ref[...]
        hrb = hr_ref[...]
        al = jnp.sum(hb * hlb, axis=-1, keepdims=True)
        w0 = _leaky(al + jnp.sum(hb * hrb, axis=-1, keepdims=True))
        w1 = _leaky(al + jnp.sum(b1 * hrb, axis=-1, keepdims=True))
        w2 = _leaky(al + jnp.sum(b2 * hrb, axis=-1, keepdims=True))
        w3 = _leaky(al + jnp.sum(b3 * hrb, axis=-1, keepdims=True))
        m = jnp.maximum(jnp.maximum(w0, w1), jnp.maximum(w2, w3))
        e0 = jnp.exp(w0 - m)
        e1 = jnp.exp(w1 - m)
        e2 = jnp.exp(w2 - m)
        e3 = jnp.exp(w3 - m)
        s = e0 + e1 + e2 + e3
        o_ref[...] = (hb * e0 + b1 * e1 + b2 * e2 + b3 * e3) / s

    spec = pl.BlockSpec((RB, F), lambda i: (i, 0))
    vspec = pl.BlockSpec((1, F), lambda i: (0, 0))
    return pl.pallas_call(
        body,
        grid=(NPAD // RB,),
        in_specs=[spec, spec, spec, spec, spec, vspec, vspec],
        out_specs=spec,
        out_shape=jax.ShapeDtypeStruct((NPAD, F), f32),
    )(h, y1, y2, y3a, y3b, hl, hr)


def kernel(feat, edge_index, W, attn_l, attn_r, hop_attn_l, hop_attn_r, sigma):
    src = edge_index[0]
    dst = edge_index[1]
    feat_pad = jnp.pad(feat, ((0, NPAD - N), (0, 0)))

    alv = attn_l.reshape(F)
    arv = attn_r.reshape(F)
    wattn = jnp.tile(jnp.stack([alv, arv], axis=1), (1, F // 2))

    h, scal = _tc_prologue(feat_pad, W, wattn)
    el = scal[:, 0]
    er = scal[:, 1]

    zvec = jnp.zeros((NPAD,), f32)
    zmat = jnp.zeros((NPAD, F), f32)

    srcC = src.reshape(NC * NS * NCH, C)
    dstC = dst.reshape(NC * NS * NCH, C)
    src2 = src.reshape(NC * NS, EPT)
    dst2 = dst.reshape(NC * NS, EPT)

    ex, pss, psd, pds, pdd = _sc_edge_pass(el, er, srcC, dstC, zvec)

    rs = (NC * (NPAD // F), F)
    ssrc, sdst, no, ni = _tc_mid(pss.reshape(rs), psd.reshape(rs),
                                 pds.reshape(rs), pdd.reshape(rs))

    p, q, r = _sc_coeff_gather(ex.reshape(NC * NS, EPT), src2, dst2,
                               ssrc.reshape(NPAD), sdst.reshape(NPAD),
                               no.reshape(NPAD), ni.reshape(NPAD))

    er_ = E // F
    ahat = _tc_coeff(p.reshape(er_, F), q.reshape(er_, F),
                     r.reshape(er_, F), sigma.reshape(1, 1)).reshape(E)

    a2 = ahat.reshape(NC * NS, EPT)

    yp1 = _sc_hop(h, srcC, dstC, a2, zmat)
    y1 = _tc_add(yp1[0], yp1[1])
    yp2 = _sc_hop(y1, srcC, dstC, a2, zmat)
    y2 = _tc_add(yp2[0], yp2[1])
    yp3 = _sc_hop(y2, srcC, dstC, a2, zmat)

    rst = _tc_epilogue(h, y1, y2, yp3[0], yp3[1],
                       hop_attn_l.reshape(1, F), hop_attn_r.reshape(1, F))
    return rst[:N].reshape(N, 1, F)


# speedup vs baseline: 28.2278x; 1.0217x over previous
import functools
import jax
import jax.numpy as jnp
from jax import lax
from jax.experimental import pallas as pl
from jax.experimental.pallas import tpu as pltpu
from jax.experimental.pallas import tpu_sc as plsc

N = 10000
E = 320000
F = 128
NPAD = 10240
NC = 2
NS = 16
EPT = E // (NC * NS)
C = 80
NCH = EPT // C
SLICE = NPAD // NS
RB = 1024
NEG = 0.2

f32 = jnp.float32

@functools.cache
def _sc_mesh():
    return plsc.VectorSubcoreMesh(core_axis_name="c", subcore_axis_name="s")


@functools.cache
def _sc_params():
    import dataclasses
    cp = pltpu.CompilerParams()
    if "needs_layout_passes" in pltpu.CompilerParams.__dataclass_fields__:
        cp = dataclasses.replace(cp, needs_layout_passes=False)
    return cp


def _leaky(x):
    return jnp.where(x >= 0, x, NEG * x)


def _tc_prologue(feat_pad, W, Wattn):
    def body(f_ref, w_ref, wa_ref, h_ref, s_ref):
        x = f_ref[...]
        h = lax.dot_general(x, w_ref[...], (((1,), (1,)), ((), ())),
                            preferred_element_type=f32)
        h_ref[...] = h
        s_ref[...] = jnp.dot(h, wa_ref[...], preferred_element_type=f32)

    return pl.pallas_call(
        body,
        grid=(NPAD // RB,),
        in_specs=[
            pl.BlockSpec((RB, F), lambda i: (i, 0)),
            pl.BlockSpec((F, F), lambda i: (0, 0)),
            pl.BlockSpec((F, F), lambda i: (0, 0)),
        ],
        out_specs=[
            pl.BlockSpec((RB, F), lambda i: (i, 0)),
            pl.BlockSpec((RB, F), lambda i: (i, 0)),
        ],
        out_shape=[
            jax.ShapeDtypeStruct((NPAD, F), f32),
            jax.ShapeDtypeStruct((NPAD, F), f32),
        ],
    )(feat_pad, W, Wattn)


CE = 128
NFE = EPT // CE
TAIL = EPT - NFE * CE


def _sc_edge_pass(el, er, src, dst, zvec):
    @functools.partial(
        pl.kernel,
        mesh=_sc_mesh(),
        compiler_params=_sc_params(),
        out_type=[
            jax.ShapeDtypeStruct((E,), f32),
            jax.ShapeDtypeStruct((NC, NPAD), f32),
            jax.ShapeDtypeStruct((NC, NPAD), f32),
            jax.ShapeDtypeStruct((NC, NPAD), f32),
            jax.ShapeDtypeStruct((NC, NPAD), f32),
        ],
        scratch_types=[
            pltpu.VMEM((NPAD,), f32),
            pltpu.VMEM((NPAD,), f32),
            pltpu.VMEM((CE,), jnp.int32),
            pltpu.VMEM((CE,), jnp.int32),
            pltpu.VMEM((CE,), jnp.int32),
            pltpu.VMEM((CE,), jnp.int32),
            pltpu.VMEM((EPT,), f32),
            pltpu.VMEM((CE,), f32),
            pltpu.VMEM((TAIL,), jnp.int32),
            pltpu.VMEM((TAIL,), jnp.int32),
            pltpu.VMEM((TAIL,), f32),
            pltpu.VMEM_SHARED((NPAD,), f32),
            pltpu.VMEM_SHARED((NPAD,), f32),
            pltpu.VMEM_SHARED((NPAD,), f32),
            pltpu.VMEM_SHARED((NPAD,), f32),
            pltpu.SemaphoreType.DMA,
            pltpu.SemaphoreType.DMA,
            pltpu.SemaphoreType.DMA,
            pltpu.SemaphoreType.DMA,
            pltpu.SemaphoreType.DMA,
            pltpu.SemaphoreType.DMA,
        ],
    )
    def k(el_hbm, er_hbm, src_hbm, dst_hbm, z_hbm,
          ex_hbm, oss_hbm, osd_hbm, ods_hbm, odd_hbm,
          el_t, er_t, src_v, dst_v, src_w, dst_w, ex_a, ones_v,
          st16, dt16, ones16,
          acc_ss, acc_sd, acc_ds, acc_dd,
          sa0, sb0, sa1, sb1, t0, t1):
        cid = lax.axis_index("c")
        sid = lax.axis_index("s")
        wid = cid * NS + sid
        base = wid * EPT
        r0 = sid * SLICE

        pltpu.sync_copy(el_hbm, el_t)
        pltpu.sync_copy(er_hbm, er_t)
        for acc in (acc_ss, acc_sd, acc_ds, acc_dd):
            pltpu.sync_copy(z_hbm.at[pl.ds(r0, SLICE)],
                            acc.at[pl.ds(r0, SLICE)])
        for kk in range(CE // 16):
            ones_v[pl.ds(16 * kk, 16)] = jnp.ones((16,), f32)
        ones16[...] = jnp.ones((TAIL,), f32)
        plsc.subcore_barrier()

        def iget(hbm, ch, buf, sem):
            return pltpu.make_async_copy(
                hbm.at[pl.ds(base + ch * CE, CE)], buf, sem)

        def compute(ch, sbuf, dbuf):
            @pl.loop(0, CE // 16)
            def _(kk):
                sl = pl.ds(16 * kk, 16)
                x = (plsc.load_gather(el_t, [sbuf[sl]]) +
                     plsc.load_gather(er_t, [dbuf[sl]]))
                ex_a[pl.ds(ch * CE + 16 * kk, 16)] = jnp.exp(_leaky(x))

        def streams(ch, sbuf, dbuf, sem):
            exs = ex_a.at[pl.ds(ch * CE, CE)]
            pltpu.async_copy(exs, acc_ss.at[sbuf], sem, add=True)
            pltpu.async_copy(exs, acc_sd.at[dbuf], sem, add=True)
            pltpu.async_copy(ones_v, acc_ds.at[sbuf], sem, add=True)
            pltpu.async_copy(ones_v, acc_dd.at[dbuf], sem, add=True)

        def drain(ch, sbuf, dbuf, sem):
            exs = ex_a.at[pl.ds(ch * CE, CE)]
            pltpu.make_async_copy(exs, acc_ss.at[sbuf], sem).wait()
            pltpu.make_async_copy(exs, acc_sd.at[dbuf], sem).wait()
            pltpu.make_async_copy(ones_v, acc_ds.at[sbuf], sem).wait()
            pltpu.make_async_copy(ones_v, acc_dd.at[dbuf], sem).wait()

        iget(src_hbm, 0, src_v, sa0).start()
        iget(dst_hbm, 0, dst_v, sb0).start()
        iget(src_hbm, 1, src_w, sa1).start()
        iget(dst_hbm, 1, dst_w, sb1).start()

        @pl.loop(0, NFE // 2)
        def _(i):
            c0 = 2 * i
            iget(src_hbm, c0, src_v, sa0).wait()
            iget(dst_hbm, c0, dst_v, sb0).wait()
            compute(c0, src_v, dst_v)
            streams(c0, src_v, dst_v, t0)
            iget(src_hbm, c0 + 1, src_w, sa1).wait()
            iget(dst_hbm, c0 + 1, dst_w, sb1).wait()
            compute(c0 + 1, src_w, dst_w)
            streams(c0 + 1, src_w, dst_w, t1)
            drain(c0, src_v, dst_v, t0)

            @pl.when(i < NFE // 2 - 1)
            def _():
                iget(src_hbm, c0 + 2, src_v, sa0).start()
                iget(dst_hbm, c0 + 2, dst_v, sb0).start()
            drain(c0 + 1, src_w, dst_w, t1)

            @pl.when(i < NFE // 2 - 1)
            def _():
                iget(src_hbm, c0 + 3, src_w, sa1).start()
                iget(dst_hbm, c0 + 3, dst_w, sb1).start()

        tb = base + NFE * CE
        pltpu.sync_copy(src_hbm.at[pl.ds(tb, TAIL)], st16)
        pltpu.sync_copy(dst_hbm.at[pl.ds(tb, TAIL)], dt16)
        x = (plsc.load_gather(el_t, [st16[...]]) +
             plsc.load_gather(er_t, [dt16[...]]))
        ex_a[pl.ds(NFE * CE, TAIL)] = jnp.exp(_leaky(x))
        exs = ex_a.at[pl.ds(NFE * CE, TAIL)]
        pltpu.sync_copy(exs, acc_ss.at[st16], add=True)
        pltpu.sync_copy(exs, acc_sd.at[dt16], add=True)
        pltpu.sync_copy(ones16, acc_ds.at[st16], add=True)
        pltpu.sync_copy(ones16, acc_dd.at[dt16], add=True)
        pltpu.sync_copy(ex_a, ex_hbm.at[pl.ds(base, EPT)])

        plsc.subcore_barrier()
        for acc, out in ((acc_ss, oss_hbm), (acc_sd, osd_hbm),
                         (acc_ds, ods_hbm), (acc_dd, odd_hbm)):
            pltpu.sync_copy(acc.at[pl.ds(r0, SLICE)],
                            out.at[cid, pl.ds(r0, SLICE)])

    return k(el, er, src, dst, zvec)


def _tc_mid(pss, psd, pds, pdd):
    h = NPAD // F

    def body(ss_ref, sd_ref, ds_ref, dd_ref, oss, osd, ono, oni):
        oss[...] = ss_ref[0:h, :] + ss_ref[h:2 * h, :]
        osd[...] = sd_ref[0:h, :] + sd_ref[h:2 * h, :]
        od = ds_ref[0:h, :] + ds_ref[h:2 * h, :]
        idg = dd_ref[0:h, :] + dd_ref[h:2 * h, :]
        ono[...] = lax.rsqrt(jnp.maximum(od, 1.0))
        oni[...] = jnp.sqrt(jnp.maximum(idg, 1.0))

    spec2 = pl.BlockSpec((2 * h, F), lambda: (0, 0))
    spec1 = pl.BlockSpec((h, F), lambda: (0, 0))
    return pl.pallas_call(
        body,
        grid=(),
        in_specs=[spec2] * 4,
        out_specs=[spec1] * 4,
        out_shape=[jax.ShapeDtypeStruct((h, F), f32)] * 4,
    )(pss, psd, pds, pdd)


def _sc_coeff_gather(ex2, src2, dst2, ssrc, sdst, no, ni):
    NW = NC * NS

    @functools.partial(
        pl.kernel,
        mesh=_sc_mesh(),
        compiler_params=_sc_params(),
        out_type=[jax.ShapeDtypeStruct((NW, EPT), f32)] * 3,
        scratch_types=[
            pltpu.VMEM((NPAD,), f32),
            pltpu.VMEM((NPAD,), f32),
            pltpu.VMEM((NPAD,), f32),
            pltpu.VMEM((NPAD,), f32),
            pltpu.VMEM((EPT,), jnp.int32),
            pltpu.VMEM((EPT,), jnp.int32),
            pltpu.VMEM((EPT,), f32),
            pltpu.VMEM((EPT,), f32),
            pltpu.VMEM((EPT,), f32),
            pltpu.VMEM((EPT,), f32),
        ],
    )
    def k(ex_hbm, src_hbm, dst_hbm, ss_hbm, sd_hbm, no_hbm, ni_hbm,
          p_hbm, q_hbm, r_hbm,
          ss_t, sd_t, no_t, ni_t, src_a, dst_a, ex_a, p_a, q_a, r_a):
        cid = lax.axis_index("c")
        sid = lax.axis_index("s")
        wid = cid * NS + sid

        pltpu.sync_copy(ss_hbm, ss_t)
        pltpu.sync_copy(sd_hbm, sd_t)
        pltpu.sync_copy(no_hbm, no_t)
        pltpu.sync_copy(ni_hbm, ni_t)
        pltpu.sync_copy(src_hbm.at[wid], src_a)
        pltpu.sync_copy(dst_hbm.at[wid], dst_a)
        pltpu.sync_copy(ex_hbm.at[wid], ex_a)

        @pl.loop(0, EPT // 16)
        def _(t):
            sl = pl.ds(16 * t, 16)
            s16 = src_a[sl]
            d16 = dst_a[sl]
            ex16 = ex_a[sl]
            gs = plsc.load_gather(ss_t, [s16])
            gd = plsc.load_gather(sd_t, [d16])
            p_a[sl] = jnp.maximum(ex16 / jnp.maximum(gd, 1e-20), 1e-10)
            q_a[sl] = jnp.maximum(ex16 / jnp.maximum(gs, 1e-20), 1e-10)
            r_a[sl] = plsc.load_gather(no_t, [s16]) * plsc.load_gather(ni_t, [d16])

        pltpu.sync_copy(p_a, p_hbm.at[wid])
        pltpu.sync_copy(q_a, q_hbm.at[wid])
        pltpu.sync_copy(r_a, r_hbm.at[wid])

    return k(ex2, src2, dst2, ssrc, sdst, no, ni)


def _tc_coeff(p, q, r, sig):
    def body(p_ref, q_ref, r_ref, s_ref, o_ref):
        sg = jax.nn.sigmoid(s_ref[...])
        o_ref[...] = jnp.exp(sg * jnp.log(p_ref[...]) +
                             (1.0 - sg) * jnp.log(q_ref[...])) * r_ref[...]

    rows = E // F
    spec = pl.BlockSpec((rows, F), lambda: (0, 0))
    return pl.pallas_call(
        body,
        grid=(),
        in_specs=[spec, spec, spec, pl.BlockSpec((1, 1), lambda: (0, 0))],
        out_specs=spec,
        out_shape=jax.ShapeDtypeStruct((rows, F), f32),
    )(p, q, r, sig)


def _sc_hop(x, srcC, dstC, a2, zmat):
    NB = 3

    @functools.partial(
        pl.kernel,
        mesh=_sc_mesh(),
        compiler_params=_sc_params(),
        out_type=jax.ShapeDtypeStruct((NC, NPAD, F), f32),
        scratch_types=(
            [pltpu.VMEM((C,), jnp.int32)] * NB +
            [pltpu.VMEM((C,), jnp.int32)] * NB +
            [pltpu.VMEM((EPT,), f32)] +
            [pltpu.VMEM((C, F), f32)] * NB +
            [pltpu.VMEM_SHARED((NPAD, F), f32)] +
            [pltpu.SemaphoreType.DMA] * (4 * NB)
        ),
    )
    def k(x_hbm, src_hbm, dst_hbm, a_hbm, z_hbm, y_hbm, *refs):
        srcb = refs[0:NB]
        dstb = refs[NB:2 * NB]
        a_t = refs[2 * NB]
        rows = refs[2 * NB + 1:3 * NB + 1]
        acc = refs[3 * NB + 1]
        gsem = refs[3 * NB + 2:4 * NB + 2]
        ssem = refs[4 * NB + 2:5 * NB + 2]
        isem = refs[5 * NB + 2:6 * NB + 2]
        dsem = refs[6 * NB + 2:7 * NB + 2]

        cid = lax.axis_index("c")
        sid = lax.axis_index("s")
        wid = cid * NS + sid
        r0 = sid * SLICE

        pltpu.sync_copy(a_hbm.at[wid], a_t)
        pltpu.sync_copy(z_hbm.at[pl.ds(r0, SLICE)], acc.at[pl.ds(r0, SLICE)])
        plsc.subcore_barrier()

        def scale(rws, ch):
            @pl.loop(0, C // 16)
            def _(g):
                a16 = a_t[pl.ds(ch * C + 16 * g, 16)]
                for l in range(16):
                    av = a16[l]
                    for cc in range(F // 16):
                        sl = pl.ds(cc * 16, 16)
                        rws[16 * g + l, sl] = rws[16 * g + l, sl] * av

        def sget(ch, b):
            return pltpu.make_async_copy(dst_hbm.at[wid * NCH + ch],
                                         dstb[b], dsem[b])

        def iget(ch, b):
            return pltpu.make_async_copy(src_hbm.at[wid * NCH + ch],
                                         srcb[b], isem[b])

        def gat(b):
            return pltpu.make_async_copy(x_hbm.at[srcb[b]], rows[b], gsem[b])

        def sct(b):
            return pltpu.make_async_copy(rows[b], acc.at[dstb[b]], ssem[b])

        for b in range(NB):
            iget(b, b).start()
            sget(b, b).start()
        for b in range(NB):
            iget(b, b).wait()
            pltpu.async_copy(x_hbm.at[srcb[b]], rows[b], gsem[b])

        TRI = (NCH - 2) // NB

        @pl.loop(0, TRI)
        def _(i):
            c0 = NB * i
            for b in range(NB):
                gat(b).wait()
                scale(rows[b], c0 + b)
                sget(c0 + b, b).wait()
                pltpu.async_copy(rows[b], acc.at[dstb[b]], ssem[b], add=True)
            for b in range(NB):
                nc = c0 + b + NB
                sct(b).wait()

                @pl.when(nc < NCH)
                def _():
                    iget(nc, b).start()
                    sget(nc, b).start()
                    iget(nc, b).wait()
                    pltpu.async_copy(x_hbm.at[srcb[b]], rows[b], gsem[b])

        for b, ch in ((0, NCH - 2), (1, NCH - 1)):
            gat(b).wait()
            scale(rows[b], ch)
            sget(ch, b).wait()
            pltpu.sync_copy(rows[b], acc.at[dstb[b]], add=True)

        plsc.subcore_barrier()
        pltpu.sync_copy(acc.at[pl.ds(r0, SLICE)],
                        y_hbm.at[cid, pl.ds(r0, SLICE)])

    return k(x, srcC, dstC, a2, zmat)


def _tc_add(y0, y1):
    def body(a_ref, b_ref, o_ref):
        o_ref[...] = a_ref[...] + b_ref[...]

    spec = pl.BlockSpec((RB, F), lambda i: (i, 0))
    return pl.pallas_call(
        body,
        grid=(NPAD // RB,),
        in_specs=[spec, spec],
        out_specs=spec,
        out_shape=jax.ShapeDtypeStruct((NPAD, F), f32),
    )(y0, y1)


def _tc_epilogue(h, y1, y2, y3a, y3b, hl, hr):
    def body(h_ref, y1_ref, y2_ref, y3a_ref, y3b_ref, hl_ref, hr_ref, o_ref):
        hb = h_ref[...]
        b1 = y1_ref[...]
        b2 = y2_ref[...]
        b3 = y3a_ref[...] + y3b_ref[...]
        hlb = hl_ref[...]
        hrb = hr_ref[...]
        al = jnp.sum(hb * hlb, axis=-1, keepdims=True)
        w0 = _leaky(al + jnp.sum(hb * hrb, axis=-1, keepdims=True))
        w1 = _leaky(al + jnp.sum(b1 * hrb, axis=-1, keepdims=True))
        w2 = _leaky(al + jnp.sum(b2 * hrb, axis=-1, keepdims=True))
        w3 = _leaky(al + jnp.sum(b3 * hrb, axis=-1, keepdims=True))
        m = jnp.maximum(jnp.maximum(w0, w1), jnp.maximum(w2, w3))
        e0 = jnp.exp(w0 - m)
        e1 = jnp.exp(w1 - m)
        e2 = jnp.exp(w2 - m)
        e3 = jnp.exp(w3 - m)
        s = e0 + e1 + e2 + e3
        o_ref[...] = (hb * e0 + b1 * e1 + b2 * e2 + b3 * e3) / s

    spec = pl.BlockSpec((RB, F), lambda i: (i, 0))
    vspec = pl.BlockSpec((1, F), lambda i: (0, 0))
    return pl.pallas_call(
        body,
        grid=(NPAD // RB,),
        in_specs=[spec, spec, spec, spec, spec, vspec, vspec],
        out_specs=spec,
        out_shape=jax.ShapeDtypeStruct((NPAD, F), f32),
    )(h, y1, y2, y3a, y3b, hl, hr)


def kernel(feat, edge_index, W, attn_l, attn_r, hop_attn_l, hop_attn_r, sigma):
    src = edge_index[0]
    dst = edge_index[1]
    feat_pad = jnp.pad(feat, ((0, NPAD - N), (0, 0)))

    alv = attn_l.reshape(F)
    arv = attn_r.reshape(F)
    wattn = jnp.tile(jnp.stack([alv, arv], axis=1), (1, F // 2))

    h, scal = _tc_prologue(feat_pad, W, wattn)
    el = scal[:, 0]
    er = scal[:, 1]

    zvec = jnp.zeros((NPAD,), f32)
    zmat = jnp.zeros((NPAD, F), f32)

    srcC = src.reshape(NC * NS * NCH, C)
    dstC = dst.reshape(NC * NS * NCH, C)
    src2 = src.reshape(NC * NS, EPT)
    dst2 = dst.reshape(NC * NS, EPT)

    ex, pss, psd, pds, pdd = _sc_edge_pass(el, er, src, dst, zvec)

    rs = (NC * (NPAD // F), F)
    ssrc, sdst, no, ni = _tc_mid(pss.reshape(rs), psd.reshape(rs),
                                 pds.reshape(rs), pdd.reshape(rs))

    p, q, r = _sc_coeff_gather(ex.reshape(NC * NS, EPT), src2, dst2,
                               ssrc.reshape(NPAD), sdst.reshape(NPAD),
                               no.reshape(NPAD), ni.reshape(NPAD))

    er_ = E // F
    ahat = _tc_coeff(p.reshape(er_, F), q.reshape(er_, F),
                     r.reshape(er_, F), sigma.reshape(1, 1)).reshape(E)

    a2 = ahat.reshape(NC * NS, EPT)

    yp1 = _sc_hop(h, srcC, dstC, a2, zmat)
    y1 = _tc_add(yp1[0], yp1[1])
    yp2 = _sc_hop(y1, srcC, dstC, a2, zmat)
    y2 = _tc_add(yp2[0], yp2[1])
    yp3 = _sc_hop(y2, srcC, dstC, a2, zmat)

    rst = _tc_epilogue(h, y1, y2, yp3[0], yp3[1],
                       hop_attn_l.reshape(1, F), hop_attn_r.reshape(1, F))
    return rst[:N].reshape(N, 1, F)
